# Initial kernel scaffold; baseline (speedup 1.0000x reference)
import jax
import jax.numpy as jnp
from jax.experimental import pallas as pl


def kernel(g, h, Q1w, Q1b, W1w, W1b, Q2w, Q2b, W2w, W2b, Gw, Gb, gscal, gamma_out, beta_out, gamma2, beta2):
    raise NotImplementedError("write your pallas kernel here")


# speedup vs baseline: 3.3113x; 3.3113x over previous
import functools

import jax
import jax.numpy as jnp
from jax import lax
from jax.experimental import pallas as pl
from jax.experimental.pallas import tpu as pltpu
from jax.experimental.pallas import tpu_sc as plsc

N = 10000
E = 160000
D = 256
OUT = 256
HALF = 128

BM = 1000
GRID = N // BM

NTILES = 16
EPT = E // NTILES
CH = 80
NCH = EPT // CH
NPAD = 10240
RPT = NPAD // NTILES


def _mm(a, b):
    return jnp.dot(a, b, preferred_element_type=jnp.float32)


def _k1_body(h_ref, qw_ref, qb_ref, mlo_ref, mhi_ref):
    m = jnp.maximum(_mm(h_ref[...], qw_ref[...]) + qb_ref[...], 0.0)
    mlo_ref[...] = m[:, :HALF]
    mhi_ref[...] = m[:, HALF:]


def _k1(h, qw, qb):
    return pl.pallas_call(
        _k1_body,
        grid=(GRID,),
        in_specs=[
            pl.BlockSpec((BM, D), lambda i: (i, 0)),
            pl.BlockSpec((D, D), lambda i: (0, 0)),
            pl.BlockSpec((1, D), lambda i: (0, 0)),
        ],
        out_specs=[
            pl.BlockSpec((BM, HALF), lambda i: (i, 0)),
            pl.BlockSpec((BM, HALF), lambda i: (i, 0)),
        ],
        out_shape=[
            jax.ShapeDtypeStruct((N, HALF), jnp.float32),
            jax.ShapeDtypeStruct((N, HALF), jnp.float32),
        ],
    )(h, qw, qb)


def _combine(h_ref, alo_ref, ahi_ref, c0_ref, c1_ref, w_ref, wb_ref):
    r = 1.0 / jnp.maximum(c0_ref[:, 0:1] + c1_ref[:, 0:1], 1.0)
    z = _mm(h_ref[...], w_ref[0:D, :])
    z += _mm(alo_ref[...] * r, w_ref[D:D + HALF, :])
    z += _mm(ahi_ref[...] * r, w_ref[D + HALF:2 * D, :])
    z = jnp.maximum(z + wb_ref[...], 0.0)
    nrm = jnp.sqrt(jnp.sum(z * z, axis=1, keepdims=True))
    return z / (nrm + 1e-6)


def _k2_body(h_ref, alo_ref, ahi_ref, c0_ref, c1_ref, w_ref, wb_ref,
             q2_ref, q2b_ref, h1_ref, m2lo_ref, m2hi_ref):
    h1 = _combine(h_ref, alo_ref, ahi_ref, c0_ref, c1_ref, w_ref, wb_ref)
    h1_ref[...] = h1
    m2 = jnp.maximum(_mm(h1, q2_ref[...]) + q2b_ref[...], 0.0)
    m2lo_ref[...] = m2[:, :HALF]
    m2hi_ref[...] = m2[:, HALF:]


def _k2(h, alo, ahi, c0, c1, w, wb, q2, q2b):
    return pl.pallas_call(
        _k2_body,
        grid=(GRID,),
        in_specs=[
            pl.BlockSpec((BM, D), lambda i: (i, 0)),
            pl.BlockSpec((BM, HALF), lambda i: (i, 0)),
            pl.BlockSpec((BM, HALF), lambda i: (i, 0)),
            pl.BlockSpec((BM, HALF), lambda i: (i, 0)),
            pl.BlockSpec((BM, HALF), lambda i: (i, 0)),
            pl.BlockSpec((2 * D, D), lambda i: (0, 0)),
            pl.BlockSpec((1, D), lambda i: (0, 0)),
            pl.BlockSpec((D, D), lambda i: (0, 0)),
            pl.BlockSpec((1, D), lambda i: (0, 0)),
        ],
        out_specs=[
            pl.BlockSpec((BM, D), lambda i: (i, 0)),
            pl.BlockSpec((BM, HALF), lambda i: (i, 0)),
            pl.BlockSpec((BM, HALF), lambda i: (i, 0)),
        ],
        out_shape=[
            jax.ShapeDtypeStruct((N, D), jnp.float32),
            jax.ShapeDtypeStruct((N, HALF), jnp.float32),
            jax.ShapeDtypeStruct((N, HALF), jnp.float32),
        ],
    )(h, alo, ahi, c0, c1, w, wb, q2, q2b)


def _k3_body(h1_ref, alo_ref, ahi_ref, c0_ref, c1_ref, w_ref, wb_ref,
             gw_ref, gb_ref, op_ref, st_ref):
    h2 = _combine(h1_ref, alo_ref, ahi_ref, c0_ref, c1_ref, w_ref, wb_ref)
    op = jnp.maximum(_mm(h2, gw_ref[...]) + gb_ref[...], 0.0)
    op_ref[...] = op

    @pl.when(pl.program_id(0) == 0)
    def _():
        st_ref[...] = jnp.zeros_like(st_ref)

    st_ref[0:1, :] += jnp.sum(op, axis=0, keepdims=True)
    st_ref[1:2, :] += jnp.sum(op * op, axis=0, keepdims=True)


def _k3(h1, alo, ahi, c0, c1, w, wb, gw, gb):
    return pl.pallas_call(
        _k3_body,
        grid=(GRID,),
        in_specs=[
            pl.BlockSpec((BM, D), lambda i: (i, 0)),
            pl.BlockSpec((BM, HALF), lambda i: (i, 0)),
            pl.BlockSpec((BM, HALF), lambda i: (i, 0)),
            pl.BlockSpec((BM, HALF), lambda i: (i, 0)),
            pl.BlockSpec((BM, HALF), lambda i: (i, 0)),
            pl.BlockSpec((2 * D, D), lambda i: (0, 0)),
            pl.BlockSpec((1, D), lambda i: (0, 0)),
            pl.BlockSpec((D, OUT), lambda i: (0, 0)),
            pl.BlockSpec((1, OUT), lambda i: (0, 0)),
        ],
        out_specs=[
            pl.BlockSpec((BM, OUT), lambda i: (i, 0)),
            pl.BlockSpec((2, OUT), lambda i: (0, 0)),
        ],
        out_shape=[
            jax.ShapeDtypeStruct((N, OUT), jnp.float32),
            jax.ShapeDtypeStruct((2, OUT), jnp.float32),
        ],
    )(h1, alo, ahi, c0, c1, w, wb, gw, gb)


def _k4_body(op_ref, st_ref, go_ref, bo_ref, g2_ref, b2_ref, gs_ref, out_ref):
    mu = st_ref[0:1, :] * (1.0 / N)
    var = st_ref[1:2, :] * (1.0 / N) - mu * mu
    a1 = gs_ref[0, 0] * go_ref[...] / jnp.sqrt(var + 1e-5)
    a = g2_ref[...] * a1 / jnp.sqrt(a1 * a1 * var + 1e-5)
    out_ref[...] = a * (op_ref[...] - mu) + b2_ref[...]


def _k4(op, st, go, bo, g2, b2, gs):
    return pl.pallas_call(
        _k4_body,
        grid=(GRID,),
        in_specs=[
            pl.BlockSpec((BM, OUT), lambda i: (i, 0)),
            pl.BlockSpec((2, OUT), lambda i: (0, 0)),
            pl.BlockSpec((1, OUT), lambda i: (0, 0)),
            pl.BlockSpec((1, OUT), lambda i: (0, 0)),
            pl.BlockSpec((1, OUT), lambda i: (0, 0)),
            pl.BlockSpec((1, OUT), lambda i: (0, 0)),
            pl.BlockSpec((1, 1), lambda i: (0, 0)),
        ],
        out_specs=pl.BlockSpec((BM, OUT), lambda i: (i, 0)),
        out_shape=jax.ShapeDtypeStruct((N, OUT), jnp.float32),
    )(op, st, go, bo, g2, b2, gs)


def _make_sc_agg():
    mesh = plsc.VectorSubcoreMesh(core_axis_name="c", subcore_axis_name="s")

    @functools.partial(
        pl.kernel,
        mesh=mesh,
        out_type=[
            jax.ShapeDtypeStruct((NPAD, HALF), jnp.float32),
            jax.ShapeDtypeStruct((NPAD, HALF), jnp.float32),
        ],
        scratch_types=[
            pltpu.VMEM((CH,), jnp.int32),
            pltpu.VMEM((CH,), jnp.int32),
            pltpu.VMEM((CH, HALF), jnp.float32),
            pltpu.VMEM_SHARED((NPAD, HALF), jnp.float32),
            pltpu.SemaphoreType.DMA,
        ],
    )
    def agg_kernel(mlo, mhi, src, dst, zrows, agglo, agghi,
                   sidx, didx, rows, aggsh, sem):
        c = lax.axis_index("c")
        s = lax.axis_index("s")
        rbase = s * RPT
        ebase = s * EPT
        nchunk = RPT // CH

        pltpu.sync_copy(zrows, rows)
        for j in range(nchunk):
            pltpu.sync_copy(rows, aggsh.at[pl.ds(rbase + j * CH, CH), :])
        plsc.subcore_barrier()

        def accum_from(m_hbm):
            def body(i, carry):
                off = ebase + i * CH
                pltpu.sync_copy(src.at[pl.ds(off, CH)], sidx)
                pltpu.sync_copy(dst.at[pl.ds(off, CH)], didx)
                pltpu.async_copy(m_hbm.at[sidx], rows, sem).wait()
                pltpu.sync_copy(rows, aggsh.at[didx], add=True)
                return carry
            lax.fori_loop(0, NCH, body, 0)

        @pl.when(c == 0)
        def _():
            accum_from(mlo)

        @pl.when(c == 1)
        def _():
            accum_from(mhi)

        plsc.subcore_barrier()

        def copy_out(out_hbm):
            for j in range(nchunk):
                pltpu.sync_copy(aggsh.at[pl.ds(rbase + j * CH, CH), :], rows)
                pltpu.sync_copy(rows, out_hbm.at[pl.ds(rbase + j * CH, CH), :])

        @pl.when(c == 0)
        def _():
            copy_out(agglo)

        @pl.when(c == 1)
        def _():
            copy_out(agghi)

    return agg_kernel


CH2 = 40
EPW = E // 32
NCH2 = EPW // CH2


def _make_sc_cnt():
    mesh = plsc.VectorSubcoreMesh(core_axis_name="c", subcore_axis_name="s")

    @functools.partial(
        pl.kernel,
        mesh=mesh,
        out_type=[
            jax.ShapeDtypeStruct((NPAD, HALF), jnp.float32),
            jax.ShapeDtypeStruct((NPAD, HALF), jnp.float32),
        ],
        scratch_types=[
            pltpu.VMEM((CH2,), jnp.int32),
            pltpu.VMEM((CH2, HALF), jnp.float32),
            pltpu.VMEM((CH, HALF), jnp.float32),
            pltpu.VMEM_SHARED((NPAD, HALF), jnp.float32),
        ],
    )
    def cnt_kernel(dst, ones_hbm, zrows, cnt0, cnt1, didx, onesv, buf, cntsh):
        c = lax.axis_index("c")
        s = lax.axis_index("s")
        rbase = s * RPT
        nchunk = RPT // CH

        pltpu.sync_copy(zrows, buf)
        for j in range(nchunk):
            pltpu.sync_copy(buf, cntsh.at[pl.ds(rbase + j * CH, CH), :])
        pltpu.sync_copy(ones_hbm, onesv)
        plsc.subcore_barrier()

        ebase = (c * NTILES + s) * EPW

        def body(i, carry):
            off = ebase + i * CH2
            pltpu.sync_copy(dst.at[pl.ds(off, CH2)], didx)
            pltpu.sync_copy(onesv, cntsh.at[didx], add=True)
            return carry
        lax.fori_loop(0, NCH2, body, 0)

        plsc.subcore_barrier()

        def copy_out(out_hbm):
            for j in range(nchunk):
                pltpu.sync_copy(cntsh.at[pl.ds(rbase + j * CH, CH), :], buf)
                pltpu.sync_copy(buf, out_hbm.at[pl.ds(rbase + j * CH, CH), :])

        @pl.when(c == 0)
        def _():
            copy_out(cnt0)

        @pl.when(c == 1)
        def _():
            copy_out(cnt1)

    return cnt_kernel


def _sc_agg_call(mlo, mhi, src, dst):
    zrows = jnp.zeros((CH, HALF), jnp.float32)
    return _make_sc_agg()(mlo, mhi, src, dst, zrows)


def _sc_cnt_call(dst):
    zrows = jnp.zeros((CH, HALF), jnp.float32)
    ones = jnp.ones((CH2, HALF), jnp.float32)
    return _make_sc_cnt()(dst, ones, zrows)


def kernel(g, h, Q1w, Q1b, W1w, W1b, Q2w, Q2b, W2w, W2b, Gw, Gb,
           gscal, gamma_out, beta_out, gamma2, beta2):
    src = g[0]
    dst = g[1]

    q1b = Q1b.reshape(1, D)
    w1b = W1b.reshape(1, D)
    q2b = Q2b.reshape(1, D)
    w2b = W2b.reshape(1, D)
    gb = Gb.reshape(1, OUT)

    cnt0, cnt1 = _sc_cnt_call(dst)
    m1lo, m1hi = _k1(h, Q1w, q1b)
    a1lo, a1hi = _sc_agg_call(m1lo, m1hi, src, dst)
    h1, m2lo, m2hi = _k2(h, a1lo, a1hi, cnt0, cnt1, W1w, w1b, Q2w, q2b)
    a2lo, a2hi = _sc_agg_call(m2lo, m2hi, src, dst)
    outp, st = _k3(h1, a2lo, a2hi, cnt0, cnt1, W2w, w2b, Gw, gb)
    out = _k4(outp, st,
              gamma_out.reshape(1, OUT), beta_out.reshape(1, OUT),
              gamma2.reshape(1, OUT), beta2.reshape(1, OUT),
              gscal.reshape(1, 1))
    return out


# speedup vs baseline: 5.9044x; 1.7831x over previous
import functools

import jax
import jax.numpy as jnp
from jax import lax
from jax.experimental import pallas as pl
from jax.experimental.pallas import tpu as pltpu
from jax.experimental.pallas import tpu_sc as plsc

N = 10000
E = 160000
D = 256
OUT = 256
HALF = 128

BM = 1000
GRID = N // BM

NTILES = 16
EPT = E // NTILES
CH = 80
NCH = EPT // CH
NPAD = 10240
RPT = NPAD // NTILES


def _mm(a, b):
    return jnp.dot(a, b, preferred_element_type=jnp.float32)


def _k1_body(h_ref, qw_ref, qb_ref, mlo_ref, mhi_ref):
    m = jnp.maximum(_mm(h_ref[...], qw_ref[...]) + qb_ref[...], 0.0)
    mlo_ref[...] = m[:, :HALF]
    mhi_ref[...] = m[:, HALF:]


def _k1(h, qw, qb):
    return pl.pallas_call(
        _k1_body,
        grid=(GRID,),
        in_specs=[
            pl.BlockSpec((BM, D), lambda i: (i, 0)),
            pl.BlockSpec((D, D), lambda i: (0, 0)),
            pl.BlockSpec((1, D), lambda i: (0, 0)),
        ],
        out_specs=[
            pl.BlockSpec((BM, HALF), lambda i: (i, 0)),
            pl.BlockSpec((BM, HALF), lambda i: (i, 0)),
        ],
        out_shape=[
            jax.ShapeDtypeStruct((N, HALF), jnp.float32),
            jax.ShapeDtypeStruct((N, HALF), jnp.float32),
        ],
    )(h, qw, qb)


def _combine(h_ref, alo_ref, ahi_ref, c0_ref, c1_ref, w_ref, wb_ref):
    r = 1.0 / jnp.maximum(c0_ref[:, 0:1] + c1_ref[:, 0:1], 1.0)
    z = _mm(h_ref[...], w_ref[0:D, :])
    z += _mm(alo_ref[...] * r, w_ref[D:D + HALF, :])
    z += _mm(ahi_ref[...] * r, w_ref[D + HALF:2 * D, :])
    z = jnp.maximum(z + wb_ref[...], 0.0)
    nrm = jnp.sqrt(jnp.sum(z * z, axis=1, keepdims=True))
    return z / (nrm + 1e-6)


def _k2_body(h_ref, alo_ref, ahi_ref, c0_ref, c1_ref, w_ref, wb_ref,
             q2_ref, q2b_ref, h1_ref, m2lo_ref, m2hi_ref):
    h1 = _combine(h_ref, alo_ref, ahi_ref, c0_ref, c1_ref, w_ref, wb_ref)
    h1_ref[...] = h1
    m2 = jnp.maximum(_mm(h1, q2_ref[...]) + q2b_ref[...], 0.0)
    m2lo_ref[...] = m2[:, :HALF]
    m2hi_ref[...] = m2[:, HALF:]


def _k2(h, alo, ahi, c0, c1, w, wb, q2, q2b):
    return pl.pallas_call(
        _k2_body,
        grid=(GRID,),
        in_specs=[
            pl.BlockSpec((BM, D), lambda i: (i, 0)),
            pl.BlockSpec((BM, HALF), lambda i: (i, 0)),
            pl.BlockSpec((BM, HALF), lambda i: (i, 0)),
            pl.BlockSpec((BM, HALF), lambda i: (i, 0)),
            pl.BlockSpec((BM, HALF), lambda i: (i, 0)),
            pl.BlockSpec((2 * D, D), lambda i: (0, 0)),
            pl.BlockSpec((1, D), lambda i: (0, 0)),
            pl.BlockSpec((D, D), lambda i: (0, 0)),
            pl.BlockSpec((1, D), lambda i: (0, 0)),
        ],
        out_specs=[
            pl.BlockSpec((BM, D), lambda i: (i, 0)),
            pl.BlockSpec((BM, HALF), lambda i: (i, 0)),
            pl.BlockSpec((BM, HALF), lambda i: (i, 0)),
        ],
        out_shape=[
            jax.ShapeDtypeStruct((N, D), jnp.float32),
            jax.ShapeDtypeStruct((N, HALF), jnp.float32),
            jax.ShapeDtypeStruct((N, HALF), jnp.float32),
        ],
    )(h, alo, ahi, c0, c1, w, wb, q2, q2b)


def _k3_body(h1_ref, alo_ref, ahi_ref, c0_ref, c1_ref, w_ref, wb_ref,
             gw_ref, gb_ref, op_ref, st_ref):
    h2 = _combine(h1_ref, alo_ref, ahi_ref, c0_ref, c1_ref, w_ref, wb_ref)
    op = jnp.maximum(_mm(h2, gw_ref[...]) + gb_ref[...], 0.0)
    op_ref[...] = op

    @pl.when(pl.program_id(0) == 0)
    def _():
        st_ref[...] = jnp.zeros_like(st_ref)

    st_ref[0:1, :] += jnp.sum(op, axis=0, keepdims=True)
    st_ref[1:2, :] += jnp.sum(op * op, axis=0, keepdims=True)


def _k3(h1, alo, ahi, c0, c1, w, wb, gw, gb):
    return pl.pallas_call(
        _k3_body,
        grid=(GRID,),
        in_specs=[
            pl.BlockSpec((BM, D), lambda i: (i, 0)),
            pl.BlockSpec((BM, HALF), lambda i: (i, 0)),
            pl.BlockSpec((BM, HALF), lambda i: (i, 0)),
            pl.BlockSpec((BM, HALF), lambda i: (i, 0)),
            pl.BlockSpec((BM, HALF), lambda i: (i, 0)),
            pl.BlockSpec((2 * D, D), lambda i: (0, 0)),
            pl.BlockSpec((1, D), lambda i: (0, 0)),
            pl.BlockSpec((D, OUT), lambda i: (0, 0)),
            pl.BlockSpec((1, OUT), lambda i: (0, 0)),
        ],
        out_specs=[
            pl.BlockSpec((BM, OUT), lambda i: (i, 0)),
            pl.BlockSpec((2, OUT), lambda i: (0, 0)),
        ],
        out_shape=[
            jax.ShapeDtypeStruct((N, OUT), jnp.float32),
            jax.ShapeDtypeStruct((2, OUT), jnp.float32),
        ],
    )(h1, alo, ahi, c0, c1, w, wb, gw, gb)


def _k4_body(op_ref, st_ref, go_ref, bo_ref, g2_ref, b2_ref, gs_ref, out_ref):
    mu = st_ref[0:1, :] * (1.0 / N)
    var = st_ref[1:2, :] * (1.0 / N) - mu * mu
    a1 = gs_ref[0, 0] * go_ref[...] / jnp.sqrt(var + 1e-5)
    a = g2_ref[...] * a1 / jnp.sqrt(a1 * a1 * var + 1e-5)
    out_ref[...] = a * (op_ref[...] - mu) + b2_ref[...]


def _k4(op, st, go, bo, g2, b2, gs):
    return pl.pallas_call(
        _k4_body,
        grid=(GRID,),
        in_specs=[
            pl.BlockSpec((BM, OUT), lambda i: (i, 0)),
            pl.BlockSpec((2, OUT), lambda i: (0, 0)),
            pl.BlockSpec((1, OUT), lambda i: (0, 0)),
            pl.BlockSpec((1, OUT), lambda i: (0, 0)),
            pl.BlockSpec((1, OUT), lambda i: (0, 0)),
            pl.BlockSpec((1, OUT), lambda i: (0, 0)),
            pl.BlockSpec((1, 1), lambda i: (0, 0)),
        ],
        out_specs=pl.BlockSpec((BM, OUT), lambda i: (i, 0)),
        out_shape=jax.ShapeDtypeStruct((N, OUT), jnp.float32),
    )(op, st, go, bo, g2, b2, gs)


CHA = 125
NCHA = EPT // CHA


def _make_sc_agg():
    mesh = plsc.VectorSubcoreMesh(core_axis_name="c", subcore_axis_name="s")

    @functools.partial(
        pl.kernel,
        mesh=mesh,
        out_type=[
            jax.ShapeDtypeStruct((NPAD, HALF), jnp.float32),
            jax.ShapeDtypeStruct((NPAD, HALF), jnp.float32),
        ],
        scratch_types=[
            pltpu.VMEM((NCHA, CHA), jnp.int32),
            pltpu.VMEM((CHA,), jnp.int32),
            pltpu.VMEM((CHA,), jnp.int32),
            pltpu.VMEM((CHA, HALF), jnp.float32),
            pltpu.VMEM((CHA, HALF), jnp.float32),
            pltpu.VMEM_SHARED((NPAD, HALF), jnp.float32),
            pltpu.SemaphoreType.DMA,
            pltpu.SemaphoreType.DMA,
            pltpu.SemaphoreType.DMA,
            pltpu.SemaphoreType.DMA,
            pltpu.SemaphoreType.DMA,
            pltpu.SemaphoreType.DMA,
        ],
    )
    def agg_kernel(mlo, mhi, src2, dst2, zrows, agglo, agghi,
                   dstv, srcb0, srcb1, rows0, rows1, aggsh,
                   g0, g1, s0, s1, i0, i1):
        c = lax.axis_index("c")
        s = lax.axis_index("s")
        rbase = s * RPT
        nchunk = RPT // CH
        cbase = s * NCHA

        pltpu.sync_copy(zrows, rows0.at[0:CH, :])
        for j in range(nchunk):
            pltpu.sync_copy(rows0.at[0:CH, :],
                            aggsh.at[pl.ds(rbase + j * CH, CH), :])

        pltpu.sync_copy(dst2.at[pl.ds(cbase, NCHA), :], dstv)
        plsc.subcore_barrier()

        def accum_from(m_hbm):
            def iload(j, srcb, sem):
                pltpu.async_copy(src2.at[cbase + j], srcb, sem)

            def wait_i(srcb, sem):
                pltpu.make_async_copy(src2.at[0], srcb, sem).wait()

            def gath(srcb, rows, sem):
                pltpu.async_copy(m_hbm.at[srcb], rows, sem)

            def wait_g(rows, sem):
                pltpu.make_async_copy(m_hbm.at[srcb0], rows, sem).wait()

            def scat(j, rows, sem):
                pltpu.async_copy(rows, aggsh.at[dstv.at[j]], sem, add=True)

            def wait_s(rows, sem):
                pltpu.make_async_copy(rows, aggsh.at[dstv.at[0]], sem).wait()

            pltpu.sync_copy(src2.at[cbase], srcb0)
            iload(1, srcb1, i1)
            gath(srcb0, rows0, g0)
            wait_g(rows0, g0)
            wait_i(srcb1, i1)
            gath(srcb1, rows1, g1)
            iload(2, srcb0, i0)
            scat(0, rows0, s0)

            @pl.loop(1, NCHA - 1, step=2)
            def _(j):
                wait_g(rows1, g1)
                wait_s(rows0, s0)
                wait_i(srcb0, i0)
                gath(srcb0, rows0, g0)
                iload(j + 2, srcb1, i1)
                scat(j, rows1, s1)
                wait_g(rows0, g0)
                wait_s(rows1, s1)
                wait_i(srcb1, i1)
                gath(srcb1, rows1, g1)
                iload(jnp.minimum(j + 3, NCHA - 1), srcb0, i0)
                scat(j + 1, rows0, s0)

            wait_g(rows1, g1)
            wait_s(rows0, s0)
            wait_i(srcb0, i0)
            scat(NCHA - 1, rows1, s1)
            wait_s(rows1, s1)

        @pl.when(c == 0)
        def _():
            accum_from(mlo)

        @pl.when(c == 1)
        def _():
            accum_from(mhi)

        plsc.subcore_barrier()

        def copy_out(out_hbm):
            for j in range(nchunk):
                pltpu.sync_copy(aggsh.at[pl.ds(rbase + j * CH, CH), :],
                                rows0.at[0:CH, :])
                pltpu.sync_copy(rows0.at[0:CH, :],
                                out_hbm.at[pl.ds(rbase + j * CH, CH), :])

        @pl.when(c == 0)
        def _():
            copy_out(agglo)

        @pl.when(c == 1)
        def _():
            copy_out(agghi)

    return agg_kernel


CH2 = 40
EPW = E // 32
NCH2 = EPW // CH2


def _make_sc_cnt():
    mesh = plsc.VectorSubcoreMesh(core_axis_name="c", subcore_axis_name="s")

    @functools.partial(
        pl.kernel,
        mesh=mesh,
        out_type=[
            jax.ShapeDtypeStruct((NPAD, HALF), jnp.float32),
            jax.ShapeDtypeStruct((NPAD, HALF), jnp.float32),
        ],
        scratch_types=[
            pltpu.VMEM((CH2,), jnp.int32),
            pltpu.VMEM((CH2, HALF), jnp.float32),
            pltpu.VMEM((CH, HALF), jnp.float32),
            pltpu.VMEM_SHARED((NPAD, HALF), jnp.float32),
        ],
    )
    def cnt_kernel(dst, ones_hbm, zrows, cnt0, cnt1, didx, onesv, buf, cntsh):
        c = lax.axis_index("c")
        s = lax.axis_index("s")
        rbase = s * RPT
        nchunk = RPT // CH

        pltpu.sync_copy(zrows, buf)
        for j in range(nchunk):
            pltpu.sync_copy(buf, cntsh.at[pl.ds(rbase + j * CH, CH), :])
        pltpu.sync_copy(ones_hbm, onesv)
        plsc.subcore_barrier()

        ebase = (c * NTILES + s) * EPW

        def body(i, carry):
            off = ebase + i * CH2
            pltpu.sync_copy(dst.at[pl.ds(off, CH2)], didx)
            pltpu.sync_copy(onesv, cntsh.at[didx], add=True)
            return carry
        lax.fori_loop(0, NCH2, body, 0)

        plsc.subcore_barrier()

        def copy_out(out_hbm):
            for j in range(nchunk):
                pltpu.sync_copy(cntsh.at[pl.ds(rbase + j * CH, CH), :], buf)
                pltpu.sync_copy(buf, out_hbm.at[pl.ds(rbase + j * CH, CH), :])

        @pl.when(c == 0)
        def _():
            copy_out(cnt0)

        @pl.when(c == 1)
        def _():
            copy_out(cnt1)

    return cnt_kernel


def _sc_agg_call(mlo, mhi, src, dst):
    zrows = jnp.zeros((CH, HALF), jnp.float32)
    src2 = src.reshape(E // CHA, CHA)
    dst2 = dst.reshape(E // CHA, CHA)
    return _make_sc_agg()(mlo, mhi, src2, dst2, zrows)


def _sc_cnt_call(dst):
    zrows = jnp.zeros((CH, HALF), jnp.float32)
    ones = jnp.ones((CH2, HALF), jnp.float32)
    return _make_sc_cnt()(dst, ones, zrows)


def kernel(g, h, Q1w, Q1b, W1w, W1b, Q2w, Q2b, W2w, W2b, Gw, Gb,
           gscal, gamma_out, beta_out, gamma2, beta2):
    src = g[0]
    dst = g[1]

    q1b = Q1b.reshape(1, D)
    w1b = W1b.reshape(1, D)
    q2b = Q2b.reshape(1, D)
    w2b = W2b.reshape(1, D)
    gb = Gb.reshape(1, OUT)

    cnt0, cnt1 = _sc_cnt_call(dst)
    m1lo, m1hi = _k1(h, Q1w, q1b)
    a1lo, a1hi = _sc_agg_call(m1lo, m1hi, src, dst)
    h1, m2lo, m2hi = _k2(h, a1lo, a1hi, cnt0, cnt1, W1w, w1b, Q2w, q2b)
    a2lo, a2hi = _sc_agg_call(m2lo, m2hi, src, dst)
    outp, st = _k3(h1, a2lo, a2hi, cnt0, cnt1, W2w, w2b, Gw, gb)
    out = _k4(outp, st,
              gamma_out.reshape(1, OUT), beta_out.reshape(1, OUT),
              gamma2.reshape(1, OUT), beta2.reshape(1, OUT),
              gscal.reshape(1, 1))
    return out


# speedup vs baseline: 6.6752x; 1.1305x over previous
import functools

import jax
import jax.numpy as jnp
from jax import lax
from jax.experimental import pallas as pl
from jax.experimental.pallas import tpu as pltpu
from jax.experimental.pallas import tpu_sc as plsc

N = 10000
E = 160000
D = 256
OUT = 256
HALF = 128

BM = 1000
GRID = N // BM

NTILES = 16
EPT = E // NTILES
CH = 80
NCH = EPT // CH
NPAD = 10240
RPT = NPAD // NTILES


def _mm(a, b):
    return jnp.dot(a, b, preferred_element_type=jnp.float32)


def _k1_body(h_ref, qw_ref, qb_ref, mlo_ref, mhi_ref):
    m = jnp.maximum(_mm(h_ref[...], qw_ref[...]) + qb_ref[...], 0.0)
    mlo_ref[...] = m[:, :HALF]
    mhi_ref[...] = m[:, HALF:]


def _k1(h, qw, qb):
    return pl.pallas_call(
        _k1_body,
        grid=(GRID,),
        in_specs=[
            pl.BlockSpec((BM, D), lambda i: (i, 0)),
            pl.BlockSpec((D, D), lambda i: (0, 0)),
            pl.BlockSpec((1, D), lambda i: (0, 0)),
        ],
        out_specs=[
            pl.BlockSpec((BM, HALF), lambda i: (i, 0)),
            pl.BlockSpec((BM, HALF), lambda i: (i, 0)),
        ],
        out_shape=[
            jax.ShapeDtypeStruct((N, HALF), jnp.float32),
            jax.ShapeDtypeStruct((N, HALF), jnp.float32),
        ],
    )(h, qw, qb)


def _combine(h_ref, alo_ref, ahi_ref, c0_ref, c1_ref, w_ref, wb_ref):
    r = 1.0 / jnp.maximum(c0_ref[:, 0:1] + c1_ref[:, 0:1], 1.0)
    z = _mm(h_ref[...], w_ref[0:D, :])
    z += _mm(alo_ref[...] * r, w_ref[D:D + HALF, :])
    z += _mm(ahi_ref[...] * r, w_ref[D + HALF:2 * D, :])
    z = jnp.maximum(z + wb_ref[...], 0.0)
    nrm = jnp.sqrt(jnp.sum(z * z, axis=1, keepdims=True))
    return z / (nrm + 1e-6)


def _k2_body(h_ref, alo_ref, ahi_ref, c0_ref, c1_ref, w_ref, wb_ref,
             q2_ref, q2b_ref, h1_ref, m2lo_ref, m2hi_ref):
    h1 = _combine(h_ref, alo_ref, ahi_ref, c0_ref, c1_ref, w_ref, wb_ref)
    h1_ref[...] = h1
    m2 = jnp.maximum(_mm(h1, q2_ref[...]) + q2b_ref[...], 0.0)
    m2lo_ref[...] = m2[:, :HALF]
    m2hi_ref[...] = m2[:, HALF:]


def _k2(h, alo, ahi, c0, c1, w, wb, q2, q2b):
    return pl.pallas_call(
        _k2_body,
        grid=(GRID,),
        in_specs=[
            pl.BlockSpec((BM, D), lambda i: (i, 0)),
            pl.BlockSpec((BM, HALF), lambda i: (i, 0)),
            pl.BlockSpec((BM, HALF), lambda i: (i, 0)),
            pl.BlockSpec((BM, HALF), lambda i: (i, 0)),
            pl.BlockSpec((BM, HALF), lambda i: (i, 0)),
            pl.BlockSpec((2 * D, D), lambda i: (0, 0)),
            pl.BlockSpec((1, D), lambda i: (0, 0)),
            pl.BlockSpec((D, D), lambda i: (0, 0)),
            pl.BlockSpec((1, D), lambda i: (0, 0)),
        ],
        out_specs=[
            pl.BlockSpec((BM, D), lambda i: (i, 0)),
            pl.BlockSpec((BM, HALF), lambda i: (i, 0)),
            pl.BlockSpec((BM, HALF), lambda i: (i, 0)),
        ],
        out_shape=[
            jax.ShapeDtypeStruct((N, D), jnp.float32),
            jax.ShapeDtypeStruct((N, HALF), jnp.float32),
            jax.ShapeDtypeStruct((N, HALF), jnp.float32),
        ],
    )(h, alo, ahi, c0, c1, w, wb, q2, q2b)


def _k3_body(h1_ref, alo_ref, ahi_ref, c0_ref, c1_ref, w_ref, wb_ref,
             gw_ref, gb_ref, op_ref, st_ref):
    h2 = _combine(h1_ref, alo_ref, ahi_ref, c0_ref, c1_ref, w_ref, wb_ref)
    op = jnp.maximum(_mm(h2, gw_ref[...]) + gb_ref[...], 0.0)
    op_ref[...] = op

    @pl.when(pl.program_id(0) == 0)
    def _():
        st_ref[...] = jnp.zeros_like(st_ref)

    st_ref[0:1, :] += jnp.sum(op, axis=0, keepdims=True)
    st_ref[1:2, :] += jnp.sum(op * op, axis=0, keepdims=True)


def _k3(h1, alo, ahi, c0, c1, w, wb, gw, gb):
    return pl.pallas_call(
        _k3_body,
        grid=(GRID,),
        in_specs=[
            pl.BlockSpec((BM, D), lambda i: (i, 0)),
            pl.BlockSpec((BM, HALF), lambda i: (i, 0)),
            pl.BlockSpec((BM, HALF), lambda i: (i, 0)),
            pl.BlockSpec((BM, HALF), lambda i: (i, 0)),
            pl.BlockSpec((BM, HALF), lambda i: (i, 0)),
            pl.BlockSpec((2 * D, D), lambda i: (0, 0)),
            pl.BlockSpec((1, D), lambda i: (0, 0)),
            pl.BlockSpec((D, OUT), lambda i: (0, 0)),
            pl.BlockSpec((1, OUT), lambda i: (0, 0)),
        ],
        out_specs=[
            pl.BlockSpec((BM, OUT), lambda i: (i, 0)),
            pl.BlockSpec((2, OUT), lambda i: (0, 0)),
        ],
        out_shape=[
            jax.ShapeDtypeStruct((N, OUT), jnp.float32),
            jax.ShapeDtypeStruct((2, OUT), jnp.float32),
        ],
    )(h1, alo, ahi, c0, c1, w, wb, gw, gb)


def _k4_body(op_ref, st_ref, go_ref, bo_ref, g2_ref, b2_ref, gs_ref, out_ref):
    mu = st_ref[0:1, :] * (1.0 / N)
    var = st_ref[1:2, :] * (1.0 / N) - mu * mu
    a1 = gs_ref[0, 0] * go_ref[...] / jnp.sqrt(var + 1e-5)
    a = g2_ref[...] * a1 / jnp.sqrt(a1 * a1 * var + 1e-5)
    out_ref[...] = a * (op_ref[...] - mu) + b2_ref[...]


def _k4(op, st, go, bo, g2, b2, gs):
    return pl.pallas_call(
        _k4_body,
        grid=(GRID,),
        in_specs=[
            pl.BlockSpec((BM, OUT), lambda i: (i, 0)),
            pl.BlockSpec((2, OUT), lambda i: (0, 0)),
            pl.BlockSpec((1, OUT), lambda i: (0, 0)),
            pl.BlockSpec((1, OUT), lambda i: (0, 0)),
            pl.BlockSpec((1, OUT), lambda i: (0, 0)),
            pl.BlockSpec((1, OUT), lambda i: (0, 0)),
            pl.BlockSpec((1, 1), lambda i: (0, 0)),
        ],
        out_specs=pl.BlockSpec((BM, OUT), lambda i: (i, 0)),
        out_shape=jax.ShapeDtypeStruct((N, OUT), jnp.float32),
    )(op, st, go, bo, g2, b2, gs)


CHA = 125
NCHA = EPT // CHA


def _make_sc_agg():
    mesh = plsc.VectorSubcoreMesh(core_axis_name="c", subcore_axis_name="s")

    @functools.partial(
        pl.kernel,
        mesh=mesh,
        out_type=[
            jax.ShapeDtypeStruct((NPAD, HALF), jnp.float32),
            jax.ShapeDtypeStruct((NPAD, HALF), jnp.float32),
        ],
        scratch_types=[
            pltpu.VMEM((NCHA, CHA), jnp.int32),
            pltpu.VMEM((CHA,), jnp.int32),
            pltpu.VMEM((CHA,), jnp.int32),
            pltpu.VMEM((CHA, HALF), jnp.float32),
            pltpu.VMEM((CHA, HALF), jnp.float32),
            pltpu.VMEM_SHARED((NPAD, HALF), jnp.float32),
            pltpu.SemaphoreType.DMA,
            pltpu.SemaphoreType.DMA,
            pltpu.SemaphoreType.DMA,
            pltpu.SemaphoreType.DMA,
            pltpu.SemaphoreType.DMA,
            pltpu.SemaphoreType.DMA,
        ],
    )
    def agg_kernel(mlo, mhi, src2, dst2, zrows, agglo, agghi,
                   dstv, srcb0, srcb1, rows0, rows1, aggsh,
                   g0, g1, s0, s1, i0, i1):
        c = lax.axis_index("c")
        s = lax.axis_index("s")
        rbase = s * RPT
        nchunk = RPT // CH
        cbase = s * NCHA

        pltpu.sync_copy(zrows, rows0.at[0:CH, :])
        for j in range(nchunk):
            pltpu.sync_copy(rows0.at[0:CH, :],
                            aggsh.at[pl.ds(rbase + j * CH, CH), :])

        pltpu.sync_copy(dst2.at[pl.ds(cbase, NCHA), :], dstv)
        plsc.subcore_barrier()

        def accum_from(m_hbm):
            def iload(j, srcb, sem):
                pltpu.async_copy(src2.at[cbase + j], srcb, sem)

            def wait_i(srcb, sem):
                pltpu.make_async_copy(src2.at[0], srcb, sem).wait()

            def gath(srcb, rows, sem):
                pltpu.async_copy(m_hbm.at[srcb], rows, sem)

            def wait_g(rows, sem):
                pltpu.make_async_copy(m_hbm.at[srcb0], rows, sem).wait()

            def scat(j, rows, sem):
                pltpu.async_copy(rows, aggsh.at[dstv.at[j]], sem, add=True)

            def wait_s(rows, sem):
                pltpu.make_async_copy(rows, aggsh.at[dstv.at[0]], sem).wait()

            pltpu.sync_copy(src2.at[cbase], srcb0)
            iload(1, srcb1, i1)
            gath(srcb0, rows0, g0)
            wait_g(rows0, g0)
            wait_i(srcb1, i1)
            gath(srcb1, rows1, g1)
            iload(2, srcb0, i0)
            scat(0, rows0, s0)

            @pl.loop(1, NCHA - 1, step=2)
            def _(j):
                wait_g(rows1, g1)
                wait_s(rows0, s0)
                wait_i(srcb0, i0)
                gath(srcb0, rows0, g0)
                iload(j + 2, srcb1, i1)
                scat(j, rows1, s1)
                wait_g(rows0, g0)
                wait_s(rows1, s1)
                wait_i(srcb1, i1)
                gath(srcb1, rows1, g1)
                iload(jnp.minimum(j + 3, NCHA - 1), srcb0, i0)
                scat(j + 1, rows0, s0)

            wait_g(rows1, g1)
            wait_s(rows0, s0)
            wait_i(srcb0, i0)
            scat(NCHA - 1, rows1, s1)
            wait_s(rows1, s1)

        @pl.when(c == 0)
        def _():
            accum_from(mlo)

        @pl.when(c == 1)
        def _():
            accum_from(mhi)

        plsc.subcore_barrier()

        def copy_out(out_hbm):
            for j in range(nchunk):
                pltpu.sync_copy(aggsh.at[pl.ds(rbase + j * CH, CH), :],
                                rows0.at[0:CH, :])
                pltpu.sync_copy(rows0.at[0:CH, :],
                                out_hbm.at[pl.ds(rbase + j * CH, CH), :])

        @pl.when(c == 0)
        def _():
            copy_out(agglo)

        @pl.when(c == 1)
        def _():
            copy_out(agghi)

    return agg_kernel


CHC = 125
EPW = E // 32
NCHC = EPW // CHC


def _make_sc_cnt():
    mesh = plsc.VectorSubcoreMesh(core_axis_name="c", subcore_axis_name="s")

    @functools.partial(
        pl.kernel,
        mesh=mesh,
        out_type=[
            jax.ShapeDtypeStruct((NPAD, HALF), jnp.float32),
            jax.ShapeDtypeStruct((NPAD, HALF), jnp.float32),
        ],
        scratch_types=[
            pltpu.VMEM((NCHC, CHC), jnp.int32),
            pltpu.VMEM((CHC, HALF), jnp.float32),
            pltpu.VMEM((CH, HALF), jnp.float32),
            pltpu.VMEM_SHARED((NPAD, HALF), jnp.float32),
            pltpu.SemaphoreType.DMA,
            pltpu.SemaphoreType.DMA,
        ],
    )
    def cnt_kernel(dst2, ones_hbm, zrows, cnt0, cnt1,
                   dstv, onesv, buf, cntsh, s0, s1):
        c = lax.axis_index("c")
        s = lax.axis_index("s")
        rbase = s * RPT
        nchunk = RPT // CH

        pltpu.sync_copy(zrows, buf)
        for j in range(nchunk):
            pltpu.sync_copy(buf, cntsh.at[pl.ds(rbase + j * CH, CH), :])
        pltpu.sync_copy(ones_hbm, onesv)
        cbase = (c * NTILES + s) * NCHC
        pltpu.sync_copy(dst2.at[pl.ds(cbase, NCHC), :], dstv)
        plsc.subcore_barrier()

        def scat(j, sem):
            pltpu.async_copy(onesv, cntsh.at[dstv.at[j]], sem, add=True)

        def wait_s(sem):
            pltpu.make_async_copy(onesv, cntsh.at[dstv.at[0]], sem).wait()

        scat(0, s0)
        scat(1, s1)

        @pl.loop(2, NCHC, step=2)
        def _(j):
            wait_s(s0)
            scat(j, s0)
            wait_s(s1)
            scat(j + 1, s1)

        wait_s(s0)
        wait_s(s1)

        plsc.subcore_barrier()

        def copy_out(out_hbm):
            for j in range(nchunk):
                pltpu.sync_copy(cntsh.at[pl.ds(rbase + j * CH, CH), :], buf)
                pltpu.sync_copy(buf, out_hbm.at[pl.ds(rbase + j * CH, CH), :])

        @pl.when(c == 0)
        def _():
            copy_out(cnt0)

        @pl.when(c == 1)
        def _():
            copy_out(cnt1)

    return cnt_kernel


def _sc_agg_call(mlo, mhi, src, dst):
    zrows = jnp.zeros((CH, HALF), jnp.float32)
    src2 = src.reshape(E // CHA, CHA)
    dst2 = dst.reshape(E // CHA, CHA)
    return _make_sc_agg()(mlo, mhi, src2, dst2, zrows)


def _sc_cnt_call(dst):
    zrows = jnp.zeros((CH, HALF), jnp.float32)
    ones = jnp.ones((CHC, HALF), jnp.float32)
    dst2 = dst.reshape(E // CHC, CHC)
    return _make_sc_cnt()(dst2, ones, zrows)


def kernel(g, h, Q1w, Q1b, W1w, W1b, Q2w, Q2b, W2w, W2b, Gw, Gb,
           gscal, gamma_out, beta_out, gamma2, beta2):
    src = g[0]
    dst = g[1]

    q1b = Q1b.reshape(1, D)
    w1b = W1b.reshape(1, D)
    q2b = Q2b.reshape(1, D)
    w2b = W2b.reshape(1, D)
    gb = Gb.reshape(1, OUT)

    cnt0, cnt1 = _sc_cnt_call(dst)
    m1lo, m1hi = _k1(h, Q1w, q1b)
    a1lo, a1hi = _sc_agg_call(m1lo, m1hi, src, dst)
    h1, m2lo, m2hi = _k2(h, a1lo, a1hi, cnt0, cnt1, W1w, w1b, Q2w, q2b)
    a2lo, a2hi = _sc_agg_call(m2lo, m2hi, src, dst)
    outp, st = _k3(h1, a2lo, a2hi, cnt0, cnt1, W2w, w2b, Gw, gb)
    out = _k4(outp, st,
              gamma_out.reshape(1, OUT), beta_out.reshape(1, OUT),
              gamma2.reshape(1, OUT), beta2.reshape(1, OUT),
              gscal.reshape(1, 1))
    return out


# speedup vs baseline: 6.6879x; 1.0019x over previous
import functools

import jax
import jax.numpy as jnp
from jax import lax
from jax.experimental import pallas as pl
from jax.experimental.pallas import tpu as pltpu
from jax.experimental.pallas import tpu_sc as plsc

N = 10000
E = 160000
D = 256
OUT = 256
HALF = 128

BM = 1000
GRID = N // BM

NTILES = 16
EPT = E // NTILES
CH = 80
NCH = EPT // CH
NPAD = 10112
RPT = NPAD // NTILES
ZCH = [(i * CH, CH) for i in range(RPT // CH)] + [((RPT // CH) * CH, RPT % CH)]


def _mm(a, b):
    return jnp.dot(a, b, preferred_element_type=jnp.float32)


def _k1_body(h_ref, qw_ref, qb_ref, mlo_ref, mhi_ref):
    m = jnp.maximum(_mm(h_ref[...], qw_ref[...]) + qb_ref[...], 0.0)
    mlo_ref[...] = m[:, :HALF]
    mhi_ref[...] = m[:, HALF:]


def _k1(h, qw, qb):
    return pl.pallas_call(
        _k1_body,
        grid=(GRID,),
        in_specs=[
            pl.BlockSpec((BM, D), lambda i: (i, 0)),
            pl.BlockSpec((D, D), lambda i: (0, 0)),
            pl.BlockSpec((1, D), lambda i: (0, 0)),
        ],
        out_specs=[
            pl.BlockSpec((BM, HALF), lambda i: (i, 0)),
            pl.BlockSpec((BM, HALF), lambda i: (i, 0)),
        ],
        out_shape=[
            jax.ShapeDtypeStruct((N, HALF), jnp.float32),
            jax.ShapeDtypeStruct((N, HALF), jnp.float32),
        ],
    )(h, qw, qb)


def _combine(h_ref, alo_ref, ahi_ref, c0_ref, c1_ref, w_ref, wb_ref):
    r = 1.0 / jnp.maximum(c0_ref[:, 0:1] + c1_ref[:, 0:1], 1.0)
    z = _mm(h_ref[...], w_ref[0:D, :])
    z += _mm(alo_ref[...] * r, w_ref[D:D + HALF, :])
    z += _mm(ahi_ref[...] * r, w_ref[D + HALF:2 * D, :])
    z = jnp.maximum(z + wb_ref[...], 0.0)
    nrm = jnp.sqrt(jnp.sum(z * z, axis=1, keepdims=True))
    return z / (nrm + 1e-6)


def _k2_body(h_ref, alo_ref, ahi_ref, c0_ref, c1_ref, w_ref, wb_ref,
             q2_ref, q2b_ref, h1_ref, m2lo_ref, m2hi_ref):
    h1 = _combine(h_ref, alo_ref, ahi_ref, c0_ref, c1_ref, w_ref, wb_ref)
    h1_ref[...] = h1
    m2 = jnp.maximum(_mm(h1, q2_ref[...]) + q2b_ref[...], 0.0)
    m2lo_ref[...] = m2[:, :HALF]
    m2hi_ref[...] = m2[:, HALF:]


def _k2(h, alo, ahi, c0, c1, w, wb, q2, q2b):
    return pl.pallas_call(
        _k2_body,
        grid=(GRID,),
        in_specs=[
            pl.BlockSpec((BM, D), lambda i: (i, 0)),
            pl.BlockSpec((BM, HALF), lambda i: (i, 0)),
            pl.BlockSpec((BM, HALF), lambda i: (i, 0)),
            pl.BlockSpec((BM, HALF), lambda i: (i, 0)),
            pl.BlockSpec((BM, HALF), lambda i: (i, 0)),
            pl.BlockSpec((2 * D, D), lambda i: (0, 0)),
            pl.BlockSpec((1, D), lambda i: (0, 0)),
            pl.BlockSpec((D, D), lambda i: (0, 0)),
            pl.BlockSpec((1, D), lambda i: (0, 0)),
        ],
        out_specs=[
            pl.BlockSpec((BM, D), lambda i: (i, 0)),
            pl.BlockSpec((BM, HALF), lambda i: (i, 0)),
            pl.BlockSpec((BM, HALF), lambda i: (i, 0)),
        ],
        out_shape=[
            jax.ShapeDtypeStruct((N, D), jnp.float32),
            jax.ShapeDtypeStruct((N, HALF), jnp.float32),
            jax.ShapeDtypeStruct((N, HALF), jnp.float32),
        ],
    )(h, alo, ahi, c0, c1, w, wb, q2, q2b)


def _k3_body(h1_ref, alo_ref, ahi_ref, c0_ref, c1_ref, w_ref, wb_ref,
             gw_ref, gb_ref, op_ref, st_ref):
    h2 = _combine(h1_ref, alo_ref, ahi_ref, c0_ref, c1_ref, w_ref, wb_ref)
    op = jnp.maximum(_mm(h2, gw_ref[...]) + gb_ref[...], 0.0)
    op_ref[...] = op

    @pl.when(pl.program_id(0) == 0)
    def _():
        st_ref[...] = jnp.zeros_like(st_ref)

    st_ref[0:1, :] += jnp.sum(op, axis=0, keepdims=True)
    st_ref[1:2, :] += jnp.sum(op * op, axis=0, keepdims=True)


def _k3(h1, alo, ahi, c0, c1, w, wb, gw, gb):
    return pl.pallas_call(
        _k3_body,
        grid=(GRID,),
        in_specs=[
            pl.BlockSpec((BM, D), lambda i: (i, 0)),
            pl.BlockSpec((BM, HALF), lambda i: (i, 0)),
            pl.BlockSpec((BM, HALF), lambda i: (i, 0)),
            pl.BlockSpec((BM, HALF), lambda i: (i, 0)),
            pl.BlockSpec((BM, HALF), lambda i: (i, 0)),
            pl.BlockSpec((2 * D, D), lambda i: (0, 0)),
            pl.BlockSpec((1, D), lambda i: (0, 0)),
            pl.BlockSpec((D, OUT), lambda i: (0, 0)),
            pl.BlockSpec((1, OUT), lambda i: (0, 0)),
        ],
        out_specs=[
            pl.BlockSpec((BM, OUT), lambda i: (i, 0)),
            pl.BlockSpec((2, OUT), lambda i: (0, 0)),
        ],
        out_shape=[
            jax.ShapeDtypeStruct((N, OUT), jnp.float32),
            jax.ShapeDtypeStruct((2, OUT), jnp.float32),
        ],
    )(h1, alo, ahi, c0, c1, w, wb, gw, gb)


def _k4_body(op_ref, st_ref, go_ref, bo_ref, g2_ref, b2_ref, gs_ref, out_ref):
    mu = st_ref[0:1, :] * (1.0 / N)
    var = st_ref[1:2, :] * (1.0 / N) - mu * mu
    a1 = gs_ref[0, 0] * go_ref[...] / jnp.sqrt(var + 1e-5)
    a = g2_ref[...] * a1 / jnp.sqrt(a1 * a1 * var + 1e-5)
    out_ref[...] = a * (op_ref[...] - mu) + b2_ref[...]


def _k4(op, st, go, bo, g2, b2, gs):
    return pl.pallas_call(
        _k4_body,
        grid=(GRID,),
        in_specs=[
            pl.BlockSpec((BM, OUT), lambda i: (i, 0)),
            pl.BlockSpec((2, OUT), lambda i: (0, 0)),
            pl.BlockSpec((1, OUT), lambda i: (0, 0)),
            pl.BlockSpec((1, OUT), lambda i: (0, 0)),
            pl.BlockSpec((1, OUT), lambda i: (0, 0)),
            pl.BlockSpec((1, OUT), lambda i: (0, 0)),
            pl.BlockSpec((1, 1), lambda i: (0, 0)),
        ],
        out_specs=pl.BlockSpec((BM, OUT), lambda i: (i, 0)),
        out_shape=jax.ShapeDtypeStruct((N, OUT), jnp.float32),
    )(op, st, go, bo, g2, b2, gs)


CHA = 125
NCHA = EPT // CHA


def _make_sc_agg():
    mesh = plsc.VectorSubcoreMesh(core_axis_name="c", subcore_axis_name="s")

    @functools.partial(
        pl.kernel,
        mesh=mesh,
        out_type=[
            jax.ShapeDtypeStruct((NPAD, HALF), jnp.float32),
            jax.ShapeDtypeStruct((NPAD, HALF), jnp.float32),
        ],
        scratch_types=[
            pltpu.VMEM((CHA,), jnp.int32),
            pltpu.VMEM((CHA,), jnp.int32),
            pltpu.VMEM((CHA,), jnp.int32),
            pltpu.VMEM((CHA,), jnp.int32),
            pltpu.VMEM((CHA,), jnp.int32),
            pltpu.VMEM((CHA,), jnp.int32),
            pltpu.VMEM((CHA,), jnp.int32),
            pltpu.VMEM((CHA,), jnp.int32),
            pltpu.VMEM((CHA, HALF), jnp.float32),
            pltpu.VMEM((CHA, HALF), jnp.float32),
            pltpu.VMEM((CHA, HALF), jnp.float32),
            pltpu.VMEM_SHARED((NPAD, HALF), jnp.float32),
            pltpu.SemaphoreType.DMA,
            pltpu.SemaphoreType.DMA,
            pltpu.SemaphoreType.DMA,
            pltpu.SemaphoreType.DMA,
            pltpu.SemaphoreType.DMA,
            pltpu.SemaphoreType.DMA,
            pltpu.SemaphoreType.DMA,
            pltpu.SemaphoreType.DMA,
            pltpu.SemaphoreType.DMA,
            pltpu.SemaphoreType.DMA,
            pltpu.SemaphoreType.DMA,
            pltpu.SemaphoreType.DMA,
            pltpu.SemaphoreType.DMA,
            pltpu.SemaphoreType.DMA,
        ],
    )
    def agg_kernel(mlo, mhi, src2, dst2, zrows, agglo, agghi,
                   sb0, sb1, sb2, db0, db1, db2, db3, db4, r0, r1, r2, aggsh,
                   g0, g1, g2, s0, s1, s2, si0, si1, si2,
                   di0, di1, di2, di3, di4):
        c = lax.axis_index("c")
        s = lax.axis_index("s")
        rbase = s * RPT
        cbase = s * NCHA

        srcb = [sb0, sb1, sb2]
        didx = [db0, db1, db2, db3, db4]
        rows = [r0, r1, r2]
        gsem = [g0, g1, g2]
        ssem = [s0, s1, s2]
        sisem = [si0, si1, si2]
        disem = [di0, di1, di2, di3, di4]

        pltpu.sync_copy(zrows, r0.at[0:CH, :])
        for off, sz in ZCH:
            pltpu.sync_copy(r0.at[0:sz, :],
                            aggsh.at[pl.ds(rbase + off, sz), :])
        plsc.subcore_barrier()

        def accum_from(m_hbm):
            def sload(j, b):
                pltpu.async_copy(src2.at[cbase + j], srcb[b], sisem[b])

            def wait_si(b):
                pltpu.make_async_copy(src2.at[0], srcb[b], sisem[b]).wait()

            def dload(j, b):
                pltpu.async_copy(dst2.at[cbase + j], didx[b], disem[b])

            def wait_di(b):
                pltpu.make_async_copy(dst2.at[0], didx[b], disem[b]).wait()

            def gath(sb, b):
                pltpu.async_copy(m_hbm.at[srcb[sb]], rows[b], gsem[b])

            def wait_g(b):
                pltpu.make_async_copy(m_hbm.at[srcb[0]], rows[b],
                                      gsem[b]).wait()

            def scat(db, b):
                pltpu.async_copy(rows[b], aggsh.at[didx[db]], ssem[b],
                                 add=True)

            def wait_s(b):
                pltpu.make_async_copy(rows[b], aggsh.at[didx[0]],
                                      ssem[b]).wait()

            pltpu.sync_copy(src2.at[cbase], srcb[0])
            sload(1, 1)
            sload(2, 2)
            dload(0, 0)
            dload(1, 1)
            dload(2, 2)
            gath(0, 0)
            wait_g(0)
            wait_si(1)
            gath(1, 1)
            sload(3, 0)
            dload(3, 3)
            wait_di(0)
            scat(0, 0)
            wait_g(1)
            wait_si(2)
            gath(2, 2)
            sload(4, 1)
            dload(4, 4)
            wait_di(1)
            scat(1, 1)

            @pl.loop(2, NCHA - 3, step=15)
            def _(j0):
                for u in range(15):
                    b3 = (2 + u) % 3
                    b3n = (3 + u) % 3
                    b5 = (2 + u) % 5
                    dl = u % 5
                    wait_g(b3)
                    wait_s(b3n)
                    wait_si(b3n)
                    gath(b3n, b3n)
                    sload(j0 + u + 3, b3)
                    dload(j0 + u + 3, dl)
                    wait_di(b5)
                    scat(b5, b3)

            wait_g(2)
            wait_s(0)
            wait_si(0)
            gath(0, 0)
            wait_di(2)
            scat(2, 2)
            wait_g(0)
            wait_s(1)
            wait_si(1)
            gath(1, 1)
            wait_di(3)
            scat(3, 0)
            wait_g(1)
            wait_s(2)
            wait_di(4)
            scat(4, 1)
            wait_s(0)
            wait_s(1)

        @pl.when(c == 0)
        def _():
            accum_from(mlo)

        @pl.when(c == 1)
        def _():
            accum_from(mhi)

        plsc.subcore_barrier()

        def copy_out(out_hbm):
            for off, sz in ZCH:
                pltpu.sync_copy(aggsh.at[pl.ds(rbase + off, sz), :],
                                r0.at[0:sz, :])
                pltpu.sync_copy(r0.at[0:sz, :],
                                out_hbm.at[pl.ds(rbase + off, sz), :])

        @pl.when(c == 0)
        def _():
            copy_out(agglo)

        @pl.when(c == 1)
        def _():
            copy_out(agghi)

    return agg_kernel


CHC = 125
EPW = E // 32
NCHC = EPW // CHC


def _make_sc_cnt():
    mesh = plsc.VectorSubcoreMesh(core_axis_name="c", subcore_axis_name="s")

    @functools.partial(
        pl.kernel,
        mesh=mesh,
        out_type=[
            jax.ShapeDtypeStruct((NPAD, HALF), jnp.float32),
            jax.ShapeDtypeStruct((NPAD, HALF), jnp.float32),
        ],
        scratch_types=[
            pltpu.VMEM((NCHC, CHC), jnp.int32),
            pltpu.VMEM((CHC, HALF), jnp.float32),
            pltpu.VMEM((CH, HALF), jnp.float32),
            pltpu.VMEM_SHARED((NPAD, HALF), jnp.float32),
            pltpu.SemaphoreType.DMA,
            pltpu.SemaphoreType.DMA,
        ],
    )
    def cnt_kernel(dst2, ones_hbm, zrows, cnt0, cnt1,
                   dstv, onesv, buf, cntsh, s0, s1):
        c = lax.axis_index("c")
        s = lax.axis_index("s")
        rbase = s * RPT

        pltpu.sync_copy(zrows, buf)
        for off, sz in ZCH:
            pltpu.sync_copy(buf.at[0:sz, :],
                            cntsh.at[pl.ds(rbase + off, sz), :])
        pltpu.sync_copy(ones_hbm, onesv)
        cbase = (c * NTILES + s) * NCHC
        pltpu.sync_copy(dst2.at[pl.ds(cbase, NCHC), :], dstv)
        plsc.subcore_barrier()

        def scat(j, sem):
            pltpu.async_copy(onesv, cntsh.at[dstv.at[j]], sem, add=True)

        def wait_s(sem):
            pltpu.make_async_copy(onesv, cntsh.at[dstv.at[0]], sem).wait()

        scat(0, s0)
        scat(1, s1)

        @pl.loop(2, NCHC, step=2)
        def _(j):
            wait_s(s0)
            scat(j, s0)
            wait_s(s1)
            scat(j + 1, s1)

        wait_s(s0)
        wait_s(s1)

        plsc.subcore_barrier()

        def copy_out(out_hbm):
            for off, sz in ZCH:
                pltpu.sync_copy(cntsh.at[pl.ds(rbase + off, sz), :],
                                buf.at[0:sz, :])
                pltpu.sync_copy(buf.at[0:sz, :],
                                out_hbm.at[pl.ds(rbase + off, sz), :])

        @pl.when(c == 0)
        def _():
            copy_out(cnt0)

        @pl.when(c == 1)
        def _():
            copy_out(cnt1)

    return cnt_kernel


def _sc_agg_call(mlo, mhi, src, dst):
    zrows = jnp.zeros((CH, HALF), jnp.float32)
    src2 = src.reshape(E // CHA, CHA)
    dst2 = dst.reshape(E // CHA, CHA)
    return _make_sc_agg()(mlo, mhi, src2, dst2, zrows)


def _sc_cnt_call(dst):
    zrows = jnp.zeros((CH, HALF), jnp.float32)
    ones = jnp.ones((CHC, HALF), jnp.float32)
    dst2 = dst.reshape(E // CHC, CHC)
    return _make_sc_cnt()(dst2, ones, zrows)


def kernel(g, h, Q1w, Q1b, W1w, W1b, Q2w, Q2b, W2w, W2b, Gw, Gb,
           gscal, gamma_out, beta_out, gamma2, beta2):
    src = g[0]
    dst = g[1]

    q1b = Q1b.reshape(1, D)
    w1b = W1b.reshape(1, D)
    q2b = Q2b.reshape(1, D)
    w2b = W2b.reshape(1, D)
    gb = Gb.reshape(1, OUT)

    cnt0, cnt1 = _sc_cnt_call(dst)
    m1lo, m1hi = _k1(h, Q1w, q1b)
    a1lo, a1hi = _sc_agg_call(m1lo, m1hi, src, dst)
    h1, m2lo, m2hi = _k2(h, a1lo, a1hi, cnt0, cnt1, W1w, w1b, Q2w, q2b)
    a2lo, a2hi = _sc_agg_call(m2lo, m2hi, src, dst)
    outp, st = _k3(h1, a2lo, a2hi, cnt0, cnt1, W2w, w2b, Gw, gb)
    out = _k4(outp, st,
              gamma_out.reshape(1, OUT), beta_out.reshape(1, OUT),
              gamma2.reshape(1, OUT), beta2.reshape(1, OUT),
              gscal.reshape(1, 1))
    return out


# speedup vs baseline: 6.7002x; 1.0018x over previous
import functools

import jax
import jax.numpy as jnp
from jax import lax
from jax.experimental import pallas as pl
from jax.experimental.pallas import tpu as pltpu
from jax.experimental.pallas import tpu_sc as plsc

N = 10000
E = 160000
D = 256
OUT = 256
HALF = 128

BM = 1000
GRID = N // BM

NTILES = 16
EPT = E // NTILES
CH = 80
NCH = EPT // CH
NPAD = 10112
RPT = NPAD // NTILES
ZCH = [(i * CH, CH) for i in range(RPT // CH)] + [((RPT // CH) * CH, RPT % CH)]


def _mm(a, b):
    return jnp.dot(a.astype(jnp.bfloat16), b.astype(jnp.bfloat16),
                   preferred_element_type=jnp.float32)


def _k1_body(h_ref, qw_ref, qb_ref, mlo_ref, mhi_ref):
    m = jnp.maximum(_mm(h_ref[...], qw_ref[...]) + qb_ref[...], 0.0)
    mlo_ref[...] = m[:, :HALF]
    mhi_ref[...] = m[:, HALF:]


def _k1(h, qw, qb):
    return pl.pallas_call(
        _k1_body,
        grid=(GRID,),
        in_specs=[
            pl.BlockSpec((BM, D), lambda i: (i, 0)),
            pl.BlockSpec((D, D), lambda i: (0, 0)),
            pl.BlockSpec((1, D), lambda i: (0, 0)),
        ],
        out_specs=[
            pl.BlockSpec((BM, HALF), lambda i: (i, 0)),
            pl.BlockSpec((BM, HALF), lambda i: (i, 0)),
        ],
        out_shape=[
            jax.ShapeDtypeStruct((N, HALF), jnp.float32),
            jax.ShapeDtypeStruct((N, HALF), jnp.float32),
        ],
    )(h, qw, qb)


def _combine(h_ref, alo_ref, ahi_ref, c0_ref, c1_ref, w_ref, wb_ref):
    r = 1.0 / jnp.maximum(c0_ref[:, 0:1] + c1_ref[:, 0:1], 1.0)
    z = _mm(h_ref[...], w_ref[0:D, :])
    z += _mm(alo_ref[...] * r, w_ref[D:D + HALF, :])
    z += _mm(ahi_ref[...] * r, w_ref[D + HALF:2 * D, :])
    z = jnp.maximum(z + wb_ref[...], 0.0)
    nrm = jnp.sqrt(jnp.sum(z * z, axis=1, keepdims=True))
    return z / (nrm + 1e-6)


def _k2_body(h_ref, alo_ref, ahi_ref, c0_ref, c1_ref, w_ref, wb_ref,
             q2_ref, q2b_ref, h1_ref, m2lo_ref, m2hi_ref):
    h1 = _combine(h_ref, alo_ref, ahi_ref, c0_ref, c1_ref, w_ref, wb_ref)
    h1_ref[...] = h1
    m2 = jnp.maximum(_mm(h1, q2_ref[...]) + q2b_ref[...], 0.0)
    m2lo_ref[...] = m2[:, :HALF]
    m2hi_ref[...] = m2[:, HALF:]


def _k2(h, alo, ahi, c0, c1, w, wb, q2, q2b):
    return pl.pallas_call(
        _k2_body,
        grid=(GRID,),
        in_specs=[
            pl.BlockSpec((BM, D), lambda i: (i, 0)),
            pl.BlockSpec((BM, HALF), lambda i: (i, 0)),
            pl.BlockSpec((BM, HALF), lambda i: (i, 0)),
            pl.BlockSpec((BM, HALF), lambda i: (i, 0)),
            pl.BlockSpec((BM, HALF), lambda i: (i, 0)),
            pl.BlockSpec((2 * D, D), lambda i: (0, 0)),
            pl.BlockSpec((1, D), lambda i: (0, 0)),
            pl.BlockSpec((D, D), lambda i: (0, 0)),
            pl.BlockSpec((1, D), lambda i: (0, 0)),
        ],
        out_specs=[
            pl.BlockSpec((BM, D), lambda i: (i, 0)),
            pl.BlockSpec((BM, HALF), lambda i: (i, 0)),
            pl.BlockSpec((BM, HALF), lambda i: (i, 0)),
        ],
        out_shape=[
            jax.ShapeDtypeStruct((N, D), jnp.float32),
            jax.ShapeDtypeStruct((N, HALF), jnp.float32),
            jax.ShapeDtypeStruct((N, HALF), jnp.float32),
        ],
    )(h, alo, ahi, c0, c1, w, wb, q2, q2b)


def _k3_body(h1_ref, alo_ref, ahi_ref, c0_ref, c1_ref, w_ref, wb_ref,
             gw_ref, gb_ref, op_ref, st_ref):
    h2 = _combine(h1_ref, alo_ref, ahi_ref, c0_ref, c1_ref, w_ref, wb_ref)
    op = jnp.maximum(_mm(h2, gw_ref[...]) + gb_ref[...], 0.0)
    op_ref[...] = op

    @pl.when(pl.program_id(0) == 0)
    def _():
        st_ref[...] = jnp.zeros_like(st_ref)

    st_ref[0:1, :] += jnp.sum(op, axis=0, keepdims=True)
    st_ref[1:2, :] += jnp.sum(op * op, axis=0, keepdims=True)


def _k3(h1, alo, ahi, c0, c1, w, wb, gw, gb):
    return pl.pallas_call(
        _k3_body,
        grid=(GRID,),
        in_specs=[
            pl.BlockSpec((BM, D), lambda i: (i, 0)),
            pl.BlockSpec((BM, HALF), lambda i: (i, 0)),
            pl.BlockSpec((BM, HALF), lambda i: (i, 0)),
            pl.BlockSpec((BM, HALF), lambda i: (i, 0)),
            pl.BlockSpec((BM, HALF), lambda i: (i, 0)),
            pl.BlockSpec((2 * D, D), lambda i: (0, 0)),
            pl.BlockSpec((1, D), lambda i: (0, 0)),
            pl.BlockSpec((D, OUT), lambda i: (0, 0)),
            pl.BlockSpec((1, OUT), lambda i: (0, 0)),
        ],
        out_specs=[
            pl.BlockSpec((BM, OUT), lambda i: (i, 0)),
            pl.BlockSpec((2, OUT), lambda i: (0, 0)),
        ],
        out_shape=[
            jax.ShapeDtypeStruct((N, OUT), jnp.float32),
            jax.ShapeDtypeStruct((2, OUT), jnp.float32),
        ],
    )(h1, alo, ahi, c0, c1, w, wb, gw, gb)


def _k4_body(op_ref, st_ref, go_ref, bo_ref, g2_ref, b2_ref, gs_ref, out_ref):
    mu = st_ref[0:1, :] * (1.0 / N)
    var = st_ref[1:2, :] * (1.0 / N) - mu * mu
    a1 = gs_ref[0, 0] * go_ref[...] / jnp.sqrt(var + 1e-5)
    a = g2_ref[...] * a1 / jnp.sqrt(a1 * a1 * var + 1e-5)
    out_ref[...] = a * (op_ref[...] - mu) + b2_ref[...]


def _k4(op, st, go, bo, g2, b2, gs):
    return pl.pallas_call(
        _k4_body,
        grid=(GRID,),
        in_specs=[
            pl.BlockSpec((BM, OUT), lambda i: (i, 0)),
            pl.BlockSpec((2, OUT), lambda i: (0, 0)),
            pl.BlockSpec((1, OUT), lambda i: (0, 0)),
            pl.BlockSpec((1, OUT), lambda i: (0, 0)),
            pl.BlockSpec((1, OUT), lambda i: (0, 0)),
            pl.BlockSpec((1, OUT), lambda i: (0, 0)),
            pl.BlockSpec((1, 1), lambda i: (0, 0)),
        ],
        out_specs=pl.BlockSpec((BM, OUT), lambda i: (i, 0)),
        out_shape=jax.ShapeDtypeStruct((N, OUT), jnp.float32),
    )(op, st, go, bo, g2, b2, gs)


CHA = 125
NCHA = EPT // CHA


def _make_sc_agg():
    mesh = plsc.VectorSubcoreMesh(core_axis_name="c", subcore_axis_name="s")

    @functools.partial(
        pl.kernel,
        mesh=mesh,
        out_type=[
            jax.ShapeDtypeStruct((NPAD, HALF), jnp.float32),
            jax.ShapeDtypeStruct((NPAD, HALF), jnp.float32),
        ],
        scratch_types=[
            pltpu.VMEM((CHA,), jnp.int32),
            pltpu.VMEM((CHA,), jnp.int32),
            pltpu.VMEM((CHA,), jnp.int32),
            pltpu.VMEM((CHA,), jnp.int32),
            pltpu.VMEM((CHA,), jnp.int32),
            pltpu.VMEM((CHA,), jnp.int32),
            pltpu.VMEM((CHA,), jnp.int32),
            pltpu.VMEM((CHA,), jnp.int32),
            pltpu.VMEM((CHA, HALF), jnp.float32),
            pltpu.VMEM((CHA, HALF), jnp.float32),
            pltpu.VMEM((CHA, HALF), jnp.float32),
            pltpu.VMEM_SHARED((NPAD, HALF), jnp.float32),
            pltpu.SemaphoreType.DMA,
            pltpu.SemaphoreType.DMA,
            pltpu.SemaphoreType.DMA,
            pltpu.SemaphoreType.DMA,
            pltpu.SemaphoreType.DMA,
            pltpu.SemaphoreType.DMA,
            pltpu.SemaphoreType.DMA,
            pltpu.SemaphoreType.DMA,
            pltpu.SemaphoreType.DMA,
            pltpu.SemaphoreType.DMA,
            pltpu.SemaphoreType.DMA,
            pltpu.SemaphoreType.DMA,
            pltpu.SemaphoreType.DMA,
            pltpu.SemaphoreType.DMA,
        ],
    )
    def agg_kernel(mlo, mhi, src2, dst2, zrows, agglo, agghi,
                   sb0, sb1, sb2, db0, db1, db2, db3, db4, r0, r1, r2, aggsh,
                   g0, g1, g2, s0, s1, s2, si0, si1, si2,
                   di0, di1, di2, di3, di4):
        c = lax.axis_index("c")
        s = lax.axis_index("s")
        rbase = s * RPT
        cbase = s * NCHA

        srcb = [sb0, sb1, sb2]
        didx = [db0, db1, db2, db3, db4]
        rows = [r0, r1, r2]
        gsem = [g0, g1, g2]
        ssem = [s0, s1, s2]
        sisem = [si0, si1, si2]
        disem = [di0, di1, di2, di3, di4]

        pltpu.sync_copy(zrows, r0.at[0:CH, :])
        for off, sz in ZCH:
            pltpu.sync_copy(r0.at[0:sz, :],
                            aggsh.at[pl.ds(rbase + off, sz), :])
        plsc.subcore_barrier()

        def accum_from(m_hbm):
            def sload(j, b):
                pltpu.async_copy(src2.at[cbase + j], srcb[b], sisem[b])

            def wait_si(b):
                pltpu.make_async_copy(src2.at[0], srcb[b], sisem[b]).wait()

            def dload(j, b):
                pltpu.async_copy(dst2.at[cbase + j], didx[b], disem[b])

            def wait_di(b):
                pltpu.make_async_copy(dst2.at[0], didx[b], disem[b]).wait()

            def gath(sb, b):
                pltpu.async_copy(m_hbm.at[srcb[sb]], rows[b], gsem[b])

            def wait_g(b):
                pltpu.make_async_copy(m_hbm.at[srcb[0]], rows[b],
                                      gsem[b]).wait()

            def scat(db, b):
                pltpu.async_copy(rows[b], aggsh.at[didx[db]], ssem[b],
                                 add=True)

            def wait_s(b):
                pltpu.make_async_copy(rows[b], aggsh.at[didx[0]],
                                      ssem[b]).wait()

            pltpu.sync_copy(src2.at[cbase], srcb[0])
            sload(1, 1)
            sload(2, 2)
            dload(0, 0)
            dload(1, 1)
            dload(2, 2)
            gath(0, 0)
            wait_g(0)
            wait_si(1)
            gath(1, 1)
            sload(3, 0)
            dload(3, 3)
            wait_di(0)
            scat(0, 0)
            wait_g(1)
            wait_si(2)
            gath(2, 2)
            sload(4, 1)
            dload(4, 4)
            wait_di(1)
            scat(1, 1)

            @pl.loop(2, NCHA - 3, step=15)
            def _(j0):
                for u in range(15):
                    b3 = (2 + u) % 3
                    b3n = (3 + u) % 3
                    b5 = (2 + u) % 5
                    dl = u % 5
                    wait_g(b3)
                    wait_s(b3n)
                    wait_si(b3n)
                    gath(b3n, b3n)
                    sload(j0 + u + 3, b3)
                    dload(j0 + u + 3, dl)
                    wait_di(b5)
                    scat(b5, b3)

            wait_g(2)
            wait_s(0)
            wait_si(0)
            gath(0, 0)
            wait_di(2)
            scat(2, 2)
            wait_g(0)
            wait_s(1)
            wait_si(1)
            gath(1, 1)
            wait_di(3)
            scat(3, 0)
            wait_g(1)
            wait_s(2)
            wait_di(4)
            scat(4, 1)
            wait_s(0)
            wait_s(1)

        @pl.when(c == 0)
        def _():
            accum_from(mlo)

        @pl.when(c == 1)
        def _():
            accum_from(mhi)

        plsc.subcore_barrier()

        def copy_out(out_hbm):
            for off, sz in ZCH:
                pltpu.sync_copy(aggsh.at[pl.ds(rbase + off, sz), :],
                                r0.at[0:sz, :])
                pltpu.sync_copy(r0.at[0:sz, :],
                                out_hbm.at[pl.ds(rbase + off, sz), :])

        @pl.when(c == 0)
        def _():
            copy_out(agglo)

        @pl.when(c == 1)
        def _():
            copy_out(agghi)

    return agg_kernel


CHC = 125
EPW = E // 32
NCHC = EPW // CHC


def _make_sc_cnt():
    mesh = plsc.VectorSubcoreMesh(core_axis_name="c", subcore_axis_name="s")

    @functools.partial(
        pl.kernel,
        mesh=mesh,
        out_type=[
            jax.ShapeDtypeStruct((NPAD, HALF), jnp.float32),
            jax.ShapeDtypeStruct((NPAD, HALF), jnp.float32),
        ],
        scratch_types=[
            pltpu.VMEM((NCHC, CHC), jnp.int32),
            pltpu.VMEM((CHC, HALF), jnp.float32),
            pltpu.VMEM((CH, HALF), jnp.float32),
            pltpu.VMEM_SHARED((NPAD, HALF), jnp.float32),
            pltpu.SemaphoreType.DMA,
            pltpu.SemaphoreType.DMA,
        ],
    )
    def cnt_kernel(dst2, ones_hbm, zrows, cnt0, cnt1,
                   dstv, onesv, buf, cntsh, s0, s1):
        c = lax.axis_index("c")
        s = lax.axis_index("s")
        rbase = s * RPT

        pltpu.sync_copy(zrows, buf)
        for off, sz in ZCH:
            pltpu.sync_copy(buf.at[0:sz, :],
                            cntsh.at[pl.ds(rbase + off, sz), :])
        pltpu.sync_copy(ones_hbm, onesv)
        cbase = (c * NTILES + s) * NCHC
        pltpu.sync_copy(dst2.at[pl.ds(cbase, NCHC), :], dstv)
        plsc.subcore_barrier()

        def scat(j, sem):
            pltpu.async_copy(onesv, cntsh.at[dstv.at[j]], sem, add=True)

        def wait_s(sem):
            pltpu.make_async_copy(onesv, cntsh.at[dstv.at[0]], sem).wait()

        scat(0, s0)
        scat(1, s1)

        @pl.loop(2, NCHC, step=2)
        def _(j):
            wait_s(s0)
            scat(j, s0)
            wait_s(s1)
            scat(j + 1, s1)

        wait_s(s0)
        wait_s(s1)

        plsc.subcore_barrier()

        def copy_out(out_hbm):
            for off, sz in ZCH:
                pltpu.sync_copy(cntsh.at[pl.ds(rbase + off, sz), :],
                                buf.at[0:sz, :])
                pltpu.sync_copy(buf.at[0:sz, :],
                                out_hbm.at[pl.ds(rbase + off, sz), :])

        @pl.when(c == 0)
        def _():
            copy_out(cnt0)

        @pl.when(c == 1)
        def _():
            copy_out(cnt1)

    return cnt_kernel


def _sc_agg_call(mlo, mhi, src, dst):
    zrows = jnp.zeros((CH, HALF), jnp.float32)
    src2 = src.reshape(E // CHA, CHA)
    dst2 = dst.reshape(E // CHA, CHA)
    return _make_sc_agg()(mlo, mhi, src2, dst2, zrows)


def _sc_cnt_call(dst):
    zrows = jnp.zeros((CH, HALF), jnp.float32)
    ones = jnp.ones((CHC, HALF), jnp.float32)
    dst2 = dst.reshape(E // CHC, CHC)
    return _make_sc_cnt()(dst2, ones, zrows)


def kernel(g, h, Q1w, Q1b, W1w, W1b, Q2w, Q2b, W2w, W2b, Gw, Gb,
           gscal, gamma_out, beta_out, gamma2, beta2):
    src = g[0]
    dst = g[1]

    q1b = Q1b.reshape(1, D)
    w1b = W1b.reshape(1, D)
    q2b = Q2b.reshape(1, D)
    w2b = W2b.reshape(1, D)
    gb = Gb.reshape(1, OUT)

    cnt0, cnt1 = _sc_cnt_call(dst)
    m1lo, m1hi = _k1(h, Q1w, q1b)
    a1lo, a1hi = _sc_agg_call(m1lo, m1hi, src, dst)
    h1, m2lo, m2hi = _k2(h, a1lo, a1hi, cnt0, cnt1, W1w, w1b, Q2w, q2b)
    a2lo, a2hi = _sc_agg_call(m2lo, m2hi, src, dst)
    outp, st = _k3(h1, a2lo, a2hi, cnt0, cnt1, W2w, w2b, Gw, gb)
    out = _k4(outp, st,
              gamma_out.reshape(1, OUT), beta_out.reshape(1, OUT),
              gamma2.reshape(1, OUT), beta2.reshape(1, OUT),
              gscal.reshape(1, 1))
    return out


# speedup vs baseline: 6.7809x; 1.0120x over previous
import functools

import jax
import jax.numpy as jnp
from jax import lax
from jax.experimental import pallas as pl
from jax.experimental.pallas import tpu as pltpu
from jax.experimental.pallas import tpu_sc as plsc

N = 10000
E = 160000
D = 256
OUT = 256
HALF = 128

BM = 1000
GRID = N // BM

NTILES = 16
EPT = E // NTILES
CH = 80
NCH = EPT // CH
NPAD = 10112
RPT = NPAD // NTILES
ZCH = [(i * CH, CH) for i in range(RPT // CH)] + [((RPT // CH) * CH, RPT % CH)]


def _mm(a, b):
    return jnp.dot(a.astype(jnp.bfloat16), b.astype(jnp.bfloat16),
                   preferred_element_type=jnp.float32)


def _k1_body(h_ref, qw_ref, qb_ref, mlo_ref, mhi_ref):
    m = jnp.maximum(_mm(h_ref[...], qw_ref[...]) + qb_ref[...], 0.0)
    mlo_ref[...] = m[:, :HALF]
    mhi_ref[...] = m[:, HALF:]


def _k1(h, qw, qb):
    return pl.pallas_call(
        _k1_body,
        grid=(GRID,),
        in_specs=[
            pl.BlockSpec((BM, D), lambda i: (i, 0)),
            pl.BlockSpec((D, D), lambda i: (0, 0)),
            pl.BlockSpec((1, D), lambda i: (0, 0)),
        ],
        out_specs=[
            pl.BlockSpec((BM, HALF), lambda i: (i, 0)),
            pl.BlockSpec((BM, HALF), lambda i: (i, 0)),
        ],
        out_shape=[
            jax.ShapeDtypeStruct((N, HALF), jnp.float32),
            jax.ShapeDtypeStruct((N, HALF), jnp.float32),
        ],
    )(h, qw, qb)


def _combine(h_ref, alo_ref, ahi_ref, c0_ref, c1_ref, w_ref, wb_ref):
    r = 1.0 / jnp.maximum(c0_ref[:, 0:1] + c1_ref[:, 0:1], 1.0)
    z = _mm(h_ref[...], w_ref[0:D, :])
    z += _mm(alo_ref[...] * r, w_ref[D:D + HALF, :])
    z += _mm(ahi_ref[...] * r, w_ref[D + HALF:2 * D, :])
    z = jnp.maximum(z + wb_ref[...], 0.0)
    nrm = jnp.sqrt(jnp.sum(z * z, axis=1, keepdims=True))
    return z / (nrm + 1e-6)


def _k2_body(h_ref, alo_ref, ahi_ref, c0_ref, c1_ref, w_ref, wb_ref,
             q2_ref, q2b_ref, h1_ref, m2lo_ref, m2hi_ref):
    h1 = _combine(h_ref, alo_ref, ahi_ref, c0_ref, c1_ref, w_ref, wb_ref)
    h1_ref[...] = h1
    m2 = jnp.maximum(_mm(h1, q2_ref[...]) + q2b_ref[...], 0.0)
    m2lo_ref[...] = m2[:, :HALF]
    m2hi_ref[...] = m2[:, HALF:]


def _k2(h, alo, ahi, c0, c1, w, wb, q2, q2b):
    return pl.pallas_call(
        _k2_body,
        grid=(GRID,),
        in_specs=[
            pl.BlockSpec((BM, D), lambda i: (i, 0)),
            pl.BlockSpec((BM, HALF), lambda i: (i, 0)),
            pl.BlockSpec((BM, HALF), lambda i: (i, 0)),
            pl.BlockSpec((BM, 8), lambda i: (i, 0)),
            pl.BlockSpec((BM, 8), lambda i: (i, 0)),
            pl.BlockSpec((2 * D, D), lambda i: (0, 0)),
            pl.BlockSpec((1, D), lambda i: (0, 0)),
            pl.BlockSpec((D, D), lambda i: (0, 0)),
            pl.BlockSpec((1, D), lambda i: (0, 0)),
        ],
        out_specs=[
            pl.BlockSpec((BM, D), lambda i: (i, 0)),
            pl.BlockSpec((BM, HALF), lambda i: (i, 0)),
            pl.BlockSpec((BM, HALF), lambda i: (i, 0)),
        ],
        out_shape=[
            jax.ShapeDtypeStruct((N, D), jnp.float32),
            jax.ShapeDtypeStruct((N, HALF), jnp.float32),
            jax.ShapeDtypeStruct((N, HALF), jnp.float32),
        ],
    )(h, alo, ahi, c0, c1, w, wb, q2, q2b)


def _k3_body(h1_ref, alo_ref, ahi_ref, c0_ref, c1_ref, w_ref, wb_ref,
             gw_ref, gb_ref, op_ref, st_ref):
    h2 = _combine(h1_ref, alo_ref, ahi_ref, c0_ref, c1_ref, w_ref, wb_ref)
    op = jnp.maximum(_mm(h2, gw_ref[...]) + gb_ref[...], 0.0)
    op_ref[...] = op

    @pl.when(pl.program_id(0) == 0)
    def _():
        st_ref[...] = jnp.zeros_like(st_ref)

    st_ref[0:1, :] += jnp.sum(op, axis=0, keepdims=True)
    st_ref[1:2, :] += jnp.sum(op * op, axis=0, keepdims=True)


def _k3(h1, alo, ahi, c0, c1, w, wb, gw, gb):
    return pl.pallas_call(
        _k3_body,
        grid=(GRID,),
        in_specs=[
            pl.BlockSpec((BM, D), lambda i: (i, 0)),
            pl.BlockSpec((BM, HALF), lambda i: (i, 0)),
            pl.BlockSpec((BM, HALF), lambda i: (i, 0)),
            pl.BlockSpec((BM, 8), lambda i: (i, 0)),
            pl.BlockSpec((BM, 8), lambda i: (i, 0)),
            pl.BlockSpec((2 * D, D), lambda i: (0, 0)),
            pl.BlockSpec((1, D), lambda i: (0, 0)),
            pl.BlockSpec((D, OUT), lambda i: (0, 0)),
            pl.BlockSpec((1, OUT), lambda i: (0, 0)),
        ],
        out_specs=[
            pl.BlockSpec((BM, OUT), lambda i: (i, 0)),
            pl.BlockSpec((2, OUT), lambda i: (0, 0)),
        ],
        out_shape=[
            jax.ShapeDtypeStruct((N, OUT), jnp.float32),
            jax.ShapeDtypeStruct((2, OUT), jnp.float32),
        ],
    )(h1, alo, ahi, c0, c1, w, wb, gw, gb)


def _k4_body(op_ref, st_ref, go_ref, bo_ref, g2_ref, b2_ref, gs_ref, out_ref):
    mu = st_ref[0:1, :] * (1.0 / N)
    var = st_ref[1:2, :] * (1.0 / N) - mu * mu
    a1 = gs_ref[0, 0] * go_ref[...] / jnp.sqrt(var + 1e-5)
    a = g2_ref[...] * a1 / jnp.sqrt(a1 * a1 * var + 1e-5)
    out_ref[...] = a * (op_ref[...] - mu) + b2_ref[...]


def _k4(op, st, go, bo, g2, b2, gs):
    return pl.pallas_call(
        _k4_body,
        grid=(GRID,),
        in_specs=[
            pl.BlockSpec((BM, OUT), lambda i: (i, 0)),
            pl.BlockSpec((2, OUT), lambda i: (0, 0)),
            pl.BlockSpec((1, OUT), lambda i: (0, 0)),
            pl.BlockSpec((1, OUT), lambda i: (0, 0)),
            pl.BlockSpec((1, OUT), lambda i: (0, 0)),
            pl.BlockSpec((1, OUT), lambda i: (0, 0)),
            pl.BlockSpec((1, 1), lambda i: (0, 0)),
        ],
        out_specs=pl.BlockSpec((BM, OUT), lambda i: (i, 0)),
        out_shape=jax.ShapeDtypeStruct((N, OUT), jnp.float32),
    )(op, st, go, bo, g2, b2, gs)


CHA = 125
NCHA = EPT // CHA


def _make_sc_agg():
    mesh = plsc.VectorSubcoreMesh(core_axis_name="c", subcore_axis_name="s")

    @functools.partial(
        pl.kernel,
        mesh=mesh,
        out_type=[
            jax.ShapeDtypeStruct((NPAD, HALF), jnp.float32),
            jax.ShapeDtypeStruct((NPAD, HALF), jnp.float32),
        ],
        scratch_types=[
            pltpu.VMEM((CHA,), jnp.int32),
            pltpu.VMEM((CHA,), jnp.int32),
            pltpu.VMEM((CHA,), jnp.int32),
            pltpu.VMEM((CHA,), jnp.int32),
            pltpu.VMEM((CHA,), jnp.int32),
            pltpu.VMEM((CHA,), jnp.int32),
            pltpu.VMEM((CHA,), jnp.int32),
            pltpu.VMEM((CHA,), jnp.int32),
            pltpu.VMEM((CHA, HALF), jnp.float32),
            pltpu.VMEM((CHA, HALF), jnp.float32),
            pltpu.VMEM((CHA, HALF), jnp.float32),
            pltpu.VMEM_SHARED((NPAD, HALF), jnp.float32),
            pltpu.SemaphoreType.DMA,
            pltpu.SemaphoreType.DMA,
            pltpu.SemaphoreType.DMA,
            pltpu.SemaphoreType.DMA,
            pltpu.SemaphoreType.DMA,
            pltpu.SemaphoreType.DMA,
            pltpu.SemaphoreType.DMA,
            pltpu.SemaphoreType.DMA,
            pltpu.SemaphoreType.DMA,
            pltpu.SemaphoreType.DMA,
            pltpu.SemaphoreType.DMA,
            pltpu.SemaphoreType.DMA,
            pltpu.SemaphoreType.DMA,
            pltpu.SemaphoreType.DMA,
        ],
    )
    def agg_kernel(mlo, mhi, src2, dst2, zrows, agglo, agghi,
                   sb0, sb1, sb2, db0, db1, db2, db3, db4, r0, r1, r2, aggsh,
                   g0, g1, g2, s0, s1, s2, si0, si1, si2,
                   di0, di1, di2, di3, di4):
        c = lax.axis_index("c")
        s = lax.axis_index("s")
        rbase = s * RPT
        cbase = s * NCHA

        srcb = [sb0, sb1, sb2]
        didx = [db0, db1, db2, db3, db4]
        rows = [r0, r1, r2]
        gsem = [g0, g1, g2]
        ssem = [s0, s1, s2]
        sisem = [si0, si1, si2]
        disem = [di0, di1, di2, di3, di4]

        pltpu.sync_copy(zrows, r0.at[0:CH, :])
        for k, (off, sz) in enumerate(ZCH):
            if k >= 3:
                off3, sz3 = ZCH[k - 3]
                pltpu.make_async_copy(
                    r0.at[0:sz3, :],
                    aggsh.at[pl.ds(rbase + off3, sz3), :],
                    gsem[k % 3]).wait()
            pltpu.async_copy(r0.at[0:sz, :],
                             aggsh.at[pl.ds(rbase + off, sz), :],
                             gsem[k % 3])
        for k in range(len(ZCH) - 3, len(ZCH)):
            off3, sz3 = ZCH[k]
            pltpu.make_async_copy(
                r0.at[0:sz3, :],
                aggsh.at[pl.ds(rbase + off3, sz3), :],
                gsem[k % 3]).wait()
        plsc.subcore_barrier()

        def accum_from(m_hbm):
            def sload(j, b):
                pltpu.async_copy(src2.at[cbase + j], srcb[b], sisem[b])

            def wait_si(b):
                pltpu.make_async_copy(src2.at[0], srcb[b], sisem[b]).wait()

            def dload(j, b):
                pltpu.async_copy(dst2.at[cbase + j], didx[b], disem[b])

            def wait_di(b):
                pltpu.make_async_copy(dst2.at[0], didx[b], disem[b]).wait()

            def gath(sb, b):
                pltpu.async_copy(m_hbm.at[srcb[sb]], rows[b], gsem[b])

            def wait_g(b):
                pltpu.make_async_copy(m_hbm.at[srcb[0]], rows[b],
                                      gsem[b]).wait()

            def scat(db, b):
                pltpu.async_copy(rows[b], aggsh.at[didx[db]], ssem[b],
                                 add=True)

            def wait_s(b):
                pltpu.make_async_copy(rows[b], aggsh.at[didx[0]],
                                      ssem[b]).wait()

            pltpu.sync_copy(src2.at[cbase], srcb[0])
            sload(1, 1)
            sload(2, 2)
            dload(0, 0)
            dload(1, 1)
            dload(2, 2)
            gath(0, 0)
            wait_g(0)
            wait_si(1)
            gath(1, 1)
            sload(3, 0)
            dload(3, 3)
            wait_di(0)
            scat(0, 0)
            wait_g(1)
            wait_si(2)
            gath(2, 2)
            sload(4, 1)
            dload(4, 4)
            wait_di(1)
            scat(1, 1)

            @pl.loop(2, NCHA - 3, step=15)
            def _(j0):
                for u in range(15):
                    b3 = (2 + u) % 3
                    b3n = (3 + u) % 3
                    b5 = (2 + u) % 5
                    dl = u % 5
                    wait_g(b3)
                    wait_s(b3n)
                    wait_si(b3n)
                    gath(b3n, b3n)
                    sload(j0 + u + 3, b3)
                    dload(j0 + u + 3, dl)
                    wait_di(b5)
                    scat(b5, b3)

            wait_g(2)
            wait_s(0)
            wait_si(0)
            gath(0, 0)
            wait_di(2)
            scat(2, 2)
            wait_g(0)
            wait_s(1)
            wait_si(1)
            gath(1, 1)
            wait_di(3)
            scat(3, 0)
            wait_g(1)
            wait_s(2)
            wait_di(4)
            scat(4, 1)
            wait_s(0)
            wait_s(1)

        @pl.when(c == 0)
        def _():
            accum_from(mlo)

        @pl.when(c == 1)
        def _():
            accum_from(mhi)

        plsc.subcore_barrier()

        def copy_out(out_hbm):
            rr = [r0, r1, r2]
            for k, (off, sz) in enumerate(ZCH):
                b = k % 3
                if k >= 3:
                    off3, sz3 = ZCH[k - 3]
                    pltpu.make_async_copy(
                        rr[b].at[0:sz3, :],
                        out_hbm.at[pl.ds(rbase + off3, sz3), :],
                        ssem[b]).wait()
                pltpu.sync_copy(aggsh.at[pl.ds(rbase + off, sz), :],
                                rr[b].at[0:sz, :])
                pltpu.async_copy(rr[b].at[0:sz, :],
                                 out_hbm.at[pl.ds(rbase + off, sz), :],
                                 ssem[b])
            for k in range(len(ZCH) - 3, len(ZCH)):
                off3, sz3 = ZCH[k]
                pltpu.make_async_copy(
                    rr[k % 3].at[0:sz3, :],
                    out_hbm.at[pl.ds(rbase + off3, sz3), :],
                    ssem[k % 3]).wait()

        @pl.when(c == 0)
        def _():
            copy_out(agglo)

        @pl.when(c == 1)
        def _():
            copy_out(agghi)

    return agg_kernel


CHC = 125
EPW = E // 32
NCHC = EPW // CHC


def _make_sc_cnt():
    mesh = plsc.VectorSubcoreMesh(core_axis_name="c", subcore_axis_name="s")

    @functools.partial(
        pl.kernel,
        mesh=mesh,
        out_type=[
            jax.ShapeDtypeStruct((NPAD, HALF), jnp.float32),
            jax.ShapeDtypeStruct((NPAD, HALF), jnp.float32),
        ],
        scratch_types=[
            pltpu.VMEM((NCHC, CHC), jnp.int32),
            pltpu.VMEM((CHC, HALF), jnp.float32),
            pltpu.VMEM((CH, HALF), jnp.float32),
            pltpu.VMEM_SHARED((NPAD, HALF), jnp.float32),
            pltpu.SemaphoreType.DMA,
            pltpu.SemaphoreType.DMA,
        ],
    )
    def cnt_kernel(dst2, ones_hbm, zrows, cnt0, cnt1,
                   dstv, onesv, buf, cntsh, s0, s1):
        c = lax.axis_index("c")
        s = lax.axis_index("s")
        rbase = s * RPT

        pltpu.sync_copy(zrows, buf)
        for off, sz in ZCH:
            pltpu.sync_copy(buf.at[0:sz, :],
                            cntsh.at[pl.ds(rbase + off, sz), :])
        pltpu.sync_copy(ones_hbm, onesv)
        cbase = (c * NTILES + s) * NCHC
        pltpu.sync_copy(dst2.at[pl.ds(cbase, NCHC), :], dstv)
        plsc.subcore_barrier()

        def scat(j, sem):
            pltpu.async_copy(onesv, cntsh.at[dstv.at[j]], sem, add=True)

        def wait_s(sem):
            pltpu.make_async_copy(onesv, cntsh.at[dstv.at[0]], sem).wait()

        scat(0, s0)
        scat(1, s1)

        @pl.loop(2, NCHC, step=2)
        def _(j):
            wait_s(s0)
            scat(j, s0)
            wait_s(s1)
            scat(j + 1, s1)

        wait_s(s0)
        wait_s(s1)

        plsc.subcore_barrier()

        def copy_out(out_hbm):
            for off, sz in ZCH:
                pltpu.sync_copy(cntsh.at[pl.ds(rbase + off, sz), :],
                                buf.at[0:sz, :])
                pltpu.sync_copy(buf.at[0:sz, :],
                                out_hbm.at[pl.ds(rbase + off, sz), :])

        @pl.when(c == 0)
        def _():
            copy_out(cnt0)

        @pl.when(c == 1)
        def _():
            copy_out(cnt1)

    return cnt_kernel


def _sc_agg_call(mlo, mhi, src, dst):
    zrows = jnp.zeros((CH, HALF), jnp.float32)
    src2 = src.reshape(E // CHA, CHA)
    dst2 = dst.reshape(E // CHA, CHA)
    return _make_sc_agg()(mlo, mhi, src2, dst2, zrows)


def _sc_cnt_call(dst):
    zrows = jnp.zeros((CH, HALF), jnp.float32)
    ones = jnp.ones((CHC, HALF), jnp.float32)
    dst2 = dst.reshape(E // CHC, CHC)
    return _make_sc_cnt()(dst2, ones, zrows)


def kernel(g, h, Q1w, Q1b, W1w, W1b, Q2w, Q2b, W2w, W2b, Gw, Gb,
           gscal, gamma_out, beta_out, gamma2, beta2):
    src = g[0]
    dst = g[1]

    q1b = Q1b.reshape(1, D)
    w1b = W1b.reshape(1, D)
    q2b = Q2b.reshape(1, D)
    w2b = W2b.reshape(1, D)
    gb = Gb.reshape(1, OUT)

    cnt0, cnt1 = _sc_cnt_call(dst)
    cnt0 = cnt0[:, :8]
    cnt1 = cnt1[:, :8]
    m1lo, m1hi = _k1(h, Q1w, q1b)
    a1lo, a1hi = _sc_agg_call(m1lo, m1hi, src, dst)
    h1, m2lo, m2hi = _k2(h, a1lo, a1hi, cnt0, cnt1, W1w, w1b, Q2w, q2b)
    a2lo, a2hi = _sc_agg_call(m2lo, m2hi, src, dst)
    outp, st = _k3(h1, a2lo, a2hi, cnt0, cnt1, W2w, w2b, Gw, gb)
    out = _k4(outp, st,
              gamma_out.reshape(1, OUT), beta_out.reshape(1, OUT),
              gamma2.reshape(1, OUT), beta2.reshape(1, OUT),
              gscal.reshape(1, 1))
    return out


# speedup vs baseline: 6.7963x; 1.0023x over previous
import functools

import jax
import jax.numpy as jnp
from jax import lax
from jax.experimental import pallas as pl
from jax.experimental.pallas import tpu as pltpu
from jax.experimental.pallas import tpu_sc as plsc

N = 10000
E = 160000
D = 256
OUT = 256
HALF = 128

BM = 1000
GRID = N // BM

NTILES = 16
EPT = E // NTILES
CH = 80
NCH = EPT // CH
NPAD = 10112
RPT = NPAD // NTILES
ZCH = [(i * CH, CH) for i in range(RPT // CH)] + [((RPT // CH) * CH, RPT % CH)]


def _mm(a, b):
    return jnp.dot(a.astype(jnp.bfloat16), b.astype(jnp.bfloat16),
                   preferred_element_type=jnp.float32)


def _k1_body(h_ref, qw_ref, qb_ref, mlo_ref, mhi_ref):
    m = jnp.maximum(_mm(h_ref[...], qw_ref[...]) + qb_ref[...], 0.0)
    mlo_ref[...] = m[:, :HALF]
    mhi_ref[...] = m[:, HALF:]


def _k1(h, qw, qb):
    return pl.pallas_call(
        _k1_body,
        grid=(GRID,),
        in_specs=[
            pl.BlockSpec((BM, D), lambda i: (i, 0)),
            pl.BlockSpec((D, D), lambda i: (0, 0)),
            pl.BlockSpec((1, D), lambda i: (0, 0)),
        ],
        out_specs=[
            pl.BlockSpec((BM, HALF), lambda i: (i, 0)),
            pl.BlockSpec((BM, HALF), lambda i: (i, 0)),
        ],
        out_shape=[
            jax.ShapeDtypeStruct((N, HALF), jnp.float32),
            jax.ShapeDtypeStruct((N, HALF), jnp.float32),
        ],
    )(h, qw, qb)


def _combine(h_ref, alo_ref, ahi_ref, c0_ref, c1_ref, w_ref, wb_ref):
    r = 1.0 / jnp.maximum(c0_ref[:, 0:1] + c1_ref[:, 0:1], 1.0)
    z = _mm(h_ref[...], w_ref[0:D, :])
    z += _mm(alo_ref[...] * r, w_ref[D:D + HALF, :])
    z += _mm(ahi_ref[...] * r, w_ref[D + HALF:2 * D, :])
    z = jnp.maximum(z + wb_ref[...], 0.0)
    nrm = jnp.sqrt(jnp.sum(z * z, axis=1, keepdims=True))
    return z / (nrm + 1e-6)


def _k2_body(h_ref, alo_ref, ahi_ref, c0_ref, c1_ref, w_ref, wb_ref,
             q2_ref, q2b_ref, h1_ref, m2lo_ref, m2hi_ref):
    h1 = _combine(h_ref, alo_ref, ahi_ref, c0_ref, c1_ref, w_ref, wb_ref)
    h1_ref[...] = h1
    m2 = jnp.maximum(_mm(h1, q2_ref[...]) + q2b_ref[...], 0.0)
    m2lo_ref[...] = m2[:, :HALF]
    m2hi_ref[...] = m2[:, HALF:]


def _k2(h, alo, ahi, c0, c1, w, wb, q2, q2b):
    return pl.pallas_call(
        _k2_body,
        grid=(GRID,),
        in_specs=[
            pl.BlockSpec((BM, D), lambda i: (i, 0)),
            pl.BlockSpec((BM, HALF), lambda i: (i, 0)),
            pl.BlockSpec((BM, HALF), lambda i: (i, 0)),
            pl.BlockSpec((BM, 8), lambda i: (i, 0)),
            pl.BlockSpec((BM, 8), lambda i: (i, 0)),
            pl.BlockSpec((2 * D, D), lambda i: (0, 0)),
            pl.BlockSpec((1, D), lambda i: (0, 0)),
            pl.BlockSpec((D, D), lambda i: (0, 0)),
            pl.BlockSpec((1, D), lambda i: (0, 0)),
        ],
        out_specs=[
            pl.BlockSpec((BM, D), lambda i: (i, 0)),
            pl.BlockSpec((BM, HALF), lambda i: (i, 0)),
            pl.BlockSpec((BM, HALF), lambda i: (i, 0)),
        ],
        out_shape=[
            jax.ShapeDtypeStruct((N, D), jnp.float32),
            jax.ShapeDtypeStruct((N, HALF), jnp.float32),
            jax.ShapeDtypeStruct((N, HALF), jnp.float32),
        ],
    )(h, alo, ahi, c0, c1, w, wb, q2, q2b)


def _k3_body(h1_ref, alo_ref, ahi_ref, c0_ref, c1_ref, w_ref, wb_ref,
             gw_ref, gb_ref, op_ref, st_ref):
    h2 = _combine(h1_ref, alo_ref, ahi_ref, c0_ref, c1_ref, w_ref, wb_ref)
    op = jnp.maximum(_mm(h2, gw_ref[...]) + gb_ref[...], 0.0)
    op_ref[...] = op

    @pl.when(pl.program_id(0) == 0)
    def _():
        st_ref[...] = jnp.zeros_like(st_ref)

    st_ref[0:1, :] += jnp.sum(op, axis=0, keepdims=True)
    st_ref[1:2, :] += jnp.sum(op * op, axis=0, keepdims=True)


def _k3(h1, alo, ahi, c0, c1, w, wb, gw, gb):
    return pl.pallas_call(
        _k3_body,
        grid=(GRID,),
        in_specs=[
            pl.BlockSpec((BM, D), lambda i: (i, 0)),
            pl.BlockSpec((BM, HALF), lambda i: (i, 0)),
            pl.BlockSpec((BM, HALF), lambda i: (i, 0)),
            pl.BlockSpec((BM, 8), lambda i: (i, 0)),
            pl.BlockSpec((BM, 8), lambda i: (i, 0)),
            pl.BlockSpec((2 * D, D), lambda i: (0, 0)),
            pl.BlockSpec((1, D), lambda i: (0, 0)),
            pl.BlockSpec((D, OUT), lambda i: (0, 0)),
            pl.BlockSpec((1, OUT), lambda i: (0, 0)),
        ],
        out_specs=[
            pl.BlockSpec((BM, OUT), lambda i: (i, 0)),
            pl.BlockSpec((2, OUT), lambda i: (0, 0)),
        ],
        out_shape=[
            jax.ShapeDtypeStruct((N, OUT), jnp.float32),
            jax.ShapeDtypeStruct((2, OUT), jnp.float32),
        ],
    )(h1, alo, ahi, c0, c1, w, wb, gw, gb)


def _k4_body(op_ref, st_ref, go_ref, bo_ref, g2_ref, b2_ref, gs_ref, out_ref):
    mu = st_ref[0:1, :] * (1.0 / N)
    var = st_ref[1:2, :] * (1.0 / N) - mu * mu
    a1 = gs_ref[0, 0] * go_ref[...] / jnp.sqrt(var + 1e-5)
    a = g2_ref[...] * a1 / jnp.sqrt(a1 * a1 * var + 1e-5)
    out_ref[...] = a * (op_ref[...] - mu) + b2_ref[...]


def _k4(op, st, go, bo, g2, b2, gs):
    return pl.pallas_call(
        _k4_body,
        grid=(GRID,),
        in_specs=[
            pl.BlockSpec((BM, OUT), lambda i: (i, 0)),
            pl.BlockSpec((2, OUT), lambda i: (0, 0)),
            pl.BlockSpec((1, OUT), lambda i: (0, 0)),
            pl.BlockSpec((1, OUT), lambda i: (0, 0)),
            pl.BlockSpec((1, OUT), lambda i: (0, 0)),
            pl.BlockSpec((1, OUT), lambda i: (0, 0)),
            pl.BlockSpec((1, 1), lambda i: (0, 0)),
        ],
        out_specs=pl.BlockSpec((BM, OUT), lambda i: (i, 0)),
        out_shape=jax.ShapeDtypeStruct((N, OUT), jnp.float32),
    )(op, st, go, bo, g2, b2, gs)


CHA = 125
NCHA = EPT // CHA


def _make_sc_agg():
    mesh = plsc.VectorSubcoreMesh(core_axis_name="c", subcore_axis_name="s")

    @functools.partial(
        pl.kernel,
        mesh=mesh,
        out_type=[
            jax.ShapeDtypeStruct((NPAD, HALF), jnp.float32),
            jax.ShapeDtypeStruct((NPAD, HALF), jnp.float32),
        ],
        scratch_types=[
            pltpu.VMEM((CHA,), jnp.int32),
            pltpu.VMEM((CHA,), jnp.int32),
            pltpu.VMEM((CHA,), jnp.int32),
            pltpu.VMEM((CHA,), jnp.int32),
            pltpu.VMEM((CHA,), jnp.int32),
            pltpu.VMEM((CHA,), jnp.int32),
            pltpu.VMEM((CHA,), jnp.int32),
            pltpu.VMEM((CHA,), jnp.int32),
            pltpu.VMEM((CHA, HALF), jnp.float32),
            pltpu.VMEM((CHA, HALF), jnp.float32),
            pltpu.VMEM((CHA, HALF), jnp.float32),
            pltpu.VMEM_SHARED((NPAD, HALF), jnp.float32),
            pltpu.SemaphoreType.DMA,
            pltpu.SemaphoreType.DMA,
            pltpu.SemaphoreType.DMA,
            pltpu.SemaphoreType.DMA,
            pltpu.SemaphoreType.DMA,
            pltpu.SemaphoreType.DMA,
            pltpu.SemaphoreType.DMA,
            pltpu.SemaphoreType.DMA,
            pltpu.SemaphoreType.DMA,
            pltpu.SemaphoreType.DMA,
            pltpu.SemaphoreType.DMA,
            pltpu.SemaphoreType.DMA,
            pltpu.SemaphoreType.DMA,
            pltpu.SemaphoreType.DMA,
        ],
    )
    def agg_kernel(mlo, mhi, src2, dst2, zrows, agglo, agghi,
                   sb0, sb1, sb2, db0, db1, db2, db3, db4, r0, r1, r2, aggsh,
                   g0, g1, g2, s0, s1, s2, si0, si1, si2,
                   di0, di1, di2, di3, di4):
        c = lax.axis_index("c")
        s = lax.axis_index("s")
        rbase = s * RPT
        cbase = s * NCHA

        srcb = [sb0, sb1, sb2]
        didx = [db0, db1, db2, db3, db4]
        rows = [r0, r1, r2]
        gsem = [g0, g1, g2]
        ssem = [s0, s1, s2]
        sisem = [si0, si1, si2]
        disem = [di0, di1, di2, di3, di4]

        pltpu.sync_copy(zrows, r0.at[0:CH, :])
        for k, (off, sz) in enumerate(ZCH):
            if k >= 3:
                off3, sz3 = ZCH[k - 3]
                pltpu.make_async_copy(
                    r0.at[0:sz3, :],
                    aggsh.at[pl.ds(rbase + off3, sz3), :],
                    gsem[k % 3]).wait()
            pltpu.async_copy(r0.at[0:sz, :],
                             aggsh.at[pl.ds(rbase + off, sz), :],
                             gsem[k % 3])
        for k in range(len(ZCH) - 3, len(ZCH)):
            off3, sz3 = ZCH[k]
            pltpu.make_async_copy(
                r0.at[0:sz3, :],
                aggsh.at[pl.ds(rbase + off3, sz3), :],
                gsem[k % 3]).wait()
        plsc.subcore_barrier()

        def accum_from(m_hbm):
            def sload(j, b):
                pltpu.async_copy(src2.at[cbase + j], srcb[b], sisem[b])

            def wait_si(b):
                pltpu.make_async_copy(src2.at[0], srcb[b], sisem[b]).wait()

            def dload(j, b):
                pltpu.async_copy(dst2.at[cbase + j], didx[b], disem[b])

            def wait_di(b):
                pltpu.make_async_copy(dst2.at[0], didx[b], disem[b]).wait()

            def gath(sb, b):
                pltpu.async_copy(m_hbm.at[srcb[sb]], rows[b], gsem[b])

            def wait_g(b):
                pltpu.make_async_copy(m_hbm.at[srcb[0]], rows[b],
                                      gsem[b]).wait()

            def scat(db, b):
                pltpu.async_copy(rows[b], aggsh.at[didx[db]], ssem[b],
                                 add=True)

            def wait_s(b):
                pltpu.make_async_copy(rows[b], aggsh.at[didx[0]],
                                      ssem[b]).wait()

            pltpu.sync_copy(src2.at[cbase], srcb[0])
            sload(1, 1)
            sload(2, 2)
            dload(0, 0)
            dload(1, 1)
            dload(2, 2)
            gath(0, 0)
            wait_g(0)
            wait_si(1)
            gath(1, 1)
            sload(3, 0)
            dload(3, 3)
            wait_di(0)
            scat(0, 0)
            wait_g(1)
            wait_si(2)
            gath(2, 2)
            sload(4, 1)
            dload(4, 4)
            wait_di(1)
            scat(1, 1)

            @pl.loop(2, NCHA - 3, step=15)
            def _(j0):
                for u in range(15):
                    b3 = (2 + u) % 3
                    b3n = (3 + u) % 3
                    b5 = (2 + u) % 5
                    dl = u % 5
                    wait_g(b3)
                    wait_s(b3n)
                    wait_si(b3n)
                    gath(b3n, b3n)
                    sload(j0 + u + 3, b3)
                    dload(j0 + u + 3, dl)
                    wait_di(b5)
                    scat(b5, b3)

            wait_g(2)
            wait_s(0)
            wait_si(0)
            gath(0, 0)
            wait_di(2)
            scat(2, 2)
            wait_g(0)
            wait_s(1)
            wait_si(1)
            gath(1, 1)
            wait_di(3)
            scat(3, 0)
            wait_g(1)
            wait_s(2)
            wait_di(4)
            scat(4, 1)
            wait_s(0)
            wait_s(1)

        @pl.when(c == 0)
        def _():
            accum_from(mlo)

        @pl.when(c == 1)
        def _():
            accum_from(mhi)

        plsc.subcore_barrier()

        def copy_out(out_hbm):
            rr = [r0, r1, r2]
            for k, (off, sz) in enumerate(ZCH):
                b = k % 3
                if k >= 3:
                    off3, sz3 = ZCH[k - 3]
                    pltpu.make_async_copy(
                        rr[b].at[0:sz3, :],
                        out_hbm.at[pl.ds(rbase + off3, sz3), :],
                        ssem[b]).wait()
                pltpu.sync_copy(aggsh.at[pl.ds(rbase + off, sz), :],
                                rr[b].at[0:sz, :])
                pltpu.async_copy(rr[b].at[0:sz, :],
                                 out_hbm.at[pl.ds(rbase + off, sz), :],
                                 ssem[b])
            for k in range(len(ZCH) - 3, len(ZCH)):
                off3, sz3 = ZCH[k]
                pltpu.make_async_copy(
                    rr[k % 3].at[0:sz3, :],
                    out_hbm.at[pl.ds(rbase + off3, sz3), :],
                    ssem[k % 3]).wait()

        @pl.when(c == 0)
        def _():
            copy_out(agglo)

        @pl.when(c == 1)
        def _():
            copy_out(agghi)

    return agg_kernel


CHC = 125
EPW = E // 32
NCHC = EPW // CHC
CNTW = 128


def _make_sc_cnt():
    mesh = plsc.VectorSubcoreMesh(core_axis_name="c", subcore_axis_name="s")

    @functools.partial(
        pl.kernel,
        mesh=mesh,
        out_type=[
            jax.ShapeDtypeStruct((NPAD, CNTW), jnp.float32),
            jax.ShapeDtypeStruct((NPAD, CNTW), jnp.float32),
        ],
        scratch_types=[
            pltpu.VMEM((NCHC, CHC), jnp.int32),
            pltpu.VMEM((CHC, CNTW), jnp.float32),
            pltpu.VMEM((CH, CNTW), jnp.float32),
            pltpu.VMEM_SHARED((NPAD, CNTW), jnp.float32),
            pltpu.SemaphoreType.DMA,
            pltpu.SemaphoreType.DMA,
        ],
    )
    def cnt_kernel(dst2, ones_hbm, zrows, cnt0, cnt1,
                   dstv, onesv, buf, cntsh, s0, s1):
        c = lax.axis_index("c")
        s = lax.axis_index("s")
        rbase = s * RPT

        pltpu.sync_copy(zrows, buf)
        for off, sz in ZCH:
            pltpu.sync_copy(buf.at[0:sz, :],
                            cntsh.at[pl.ds(rbase + off, sz), :])
        pltpu.sync_copy(ones_hbm, onesv)
        cbase = (c * NTILES + s) * NCHC
        pltpu.sync_copy(dst2.at[pl.ds(cbase, NCHC), :], dstv)
        plsc.subcore_barrier()

        def scat(j, sem):
            pltpu.async_copy(onesv, cntsh.at[dstv.at[j]], sem, add=True)

        def wait_s(sem):
            pltpu.make_async_copy(onesv, cntsh.at[dstv.at[0]], sem).wait()

        scat(0, s0)
        scat(1, s1)

        @pl.loop(2, NCHC, step=2)
        def _(j):
            wait_s(s0)
            scat(j, s0)
            wait_s(s1)
            scat(j + 1, s1)

        wait_s(s0)
        wait_s(s1)

        plsc.subcore_barrier()

        def copy_out(out_hbm):
            for off, sz in ZCH:
                pltpu.sync_copy(cntsh.at[pl.ds(rbase + off, sz), :],
                                buf.at[0:sz, :])
                pltpu.sync_copy(buf.at[0:sz, :],
                                out_hbm.at[pl.ds(rbase + off, sz), :])

        @pl.when(c == 0)
        def _():
            copy_out(cnt0)

        @pl.when(c == 1)
        def _():
            copy_out(cnt1)

    return cnt_kernel


def _sc_agg_call(mlo, mhi, src, dst):
    zrows = jnp.zeros((CH, HALF), jnp.float32)
    src2 = src.reshape(E // CHA, CHA)
    dst2 = dst.reshape(E // CHA, CHA)
    return _make_sc_agg()(mlo, mhi, src2, dst2, zrows)


def _sc_cnt_call(dst):
    zrows = jnp.zeros((CH, CNTW), jnp.float32)
    ones = jnp.ones((CHC, CNTW), jnp.float32)
    dst2 = dst.reshape(E // CHC, CHC)
    return _make_sc_cnt()(dst2, ones, zrows)


def kernel(g, h, Q1w, Q1b, W1w, W1b, Q2w, Q2b, W2w, W2b, Gw, Gb,
           gscal, gamma_out, beta_out, gamma2, beta2):
    src = g[0]
    dst = g[1]

    q1b = Q1b.reshape(1, D)
    w1b = W1b.reshape(1, D)
    q2b = Q2b.reshape(1, D)
    w2b = W2b.reshape(1, D)
    gb = Gb.reshape(1, OUT)

    cnt0, cnt1 = _sc_cnt_call(dst)
    cnt0 = cnt0[:, :8]
    cnt1 = cnt1[:, :8]
    m1lo, m1hi = _k1(h, Q1w, q1b)
    a1lo, a1hi = _sc_agg_call(m1lo, m1hi, src, dst)
    h1, m2lo, m2hi = _k2(h, a1lo, a1hi, cnt0, cnt1, W1w, w1b, Q2w, q2b)
    a2lo, a2hi = _sc_agg_call(m2lo, m2hi, src, dst)
    outp, st = _k3(h1, a2lo, a2hi, cnt0, cnt1, W2w, w2b, Gw, gb)
    out = _k4(outp, st,
              gamma_out.reshape(1, OUT), beta_out.reshape(1, OUT),
              gamma2.reshape(1, OUT), beta2.reshape(1, OUT),
              gscal.reshape(1, 1))
    return out
